# Initial kernel scaffold; baseline (speedup 1.0000x reference)
import jax
import jax.numpy as jnp
from jax.experimental import pallas as pl


def kernel(x, edge_index, forward_level, backward_level, gate, forward_index, rc_pair_index, params):
    raise NotImplementedError("write your pallas kernel here")


# speedup vs baseline: 2.3831x; 2.3831x over previous
import functools

import jax
import jax.numpy as jnp
from jax import lax
from jax.experimental import pallas as pl
from jax.experimental.pallas import tpu as pltpu
from jax.experimental.pallas import tpu_sc as plsc

N = 10000
E = 160000
P = 4096
H = 128
M = 512
NPAD = 10240
TRASH = NPAD - 1
CH = 128
EPAD = E + 14 * CH
NGROUP = 14
NW = 32
BN_S = float(1.0 / (1.0 + 1e-5) ** 0.5)


def _iota16():
    return lax.iota(jnp.int32, 16)


def _sc_scalar(vec, j):
    return jnp.sum(jnp.where(_iota16() == j, vec, 0))


def _sc_group_body(msg_t, srs_r, drs_r, meta_r, agg_out,
                   meta_v, sidx, didx, rows, zbuf, aggsh, sem):
    cid = lax.axis_index("c")
    sid = lax.axis_index("s")
    w = sid * 2 + cid
    pltpu.sync_copy(meta_r, meta_v)
    mv = meta_v[...]
    es_k = _sc_scalar(mv, 0)
    nch = _sc_scalar(mv, 1)
    ns_k = _sc_scalar(mv, 2)
    noct = _sc_scalar(mv, 3)
    for r in range(8):
        for c in range(8):
            zbuf[r, pl.ds(c * 16, 16)] = jnp.zeros((16,), jnp.float32)

    def zbody(t, carry):
        j = pl.multiple_of(ns_k + (sid + t * 16) * 8, 8)
        pltpu.sync_copy(zbuf, aggsh.at[pl.ds(j, 8)])
        return carry

    lax.fori_loop(0, (noct + 15 - sid) // 16, zbody, 0)
    plsc.subcore_barrier()

    def ebody(t, carry):
        off = pl.multiple_of(es_k + (w + t * NW) * CH, 8)
        pltpu.sync_copy(srs_r.at[pl.ds(off, CH)], sidx)
        pltpu.sync_copy(drs_r.at[pl.ds(off, CH)], didx)
        pltpu.async_copy(msg_t.at[sidx], rows, sem).wait()
        pltpu.sync_copy(rows, aggsh.at[didx], add=True)
        return carry

    lax.fori_loop(0, (nch + NW - 1 - w) // NW, ebody, 0)
    plsc.subcore_barrier()

    def obody(t, carry):
        j = pl.multiple_of(ns_k + (sid + t * 16) * 8, 8)
        pltpu.sync_copy(aggsh.at[pl.ds(j, 8)],
                        agg_out.at[cid, pl.ds(j, 8)])
        return carry

    lax.fori_loop(0, (noct + 15 - sid) // 16, obody, 0)


_sc_group = functools.partial(
    pl.kernel,
    out_type=jax.ShapeDtypeStruct((2, NPAD, H), jnp.float32),
    mesh=plsc.VectorSubcoreMesh(core_axis_name="c", subcore_axis_name="s"),
    compiler_params=pltpu.CompilerParams(needs_layout_passes=False),
    scratch_types=[
        pltpu.VMEM((16,), jnp.int32),
        pltpu.VMEM((CH,), jnp.int32),
        pltpu.VMEM((CH,), jnp.int32),
        pltpu.VMEM((CH, H), jnp.float32),
        pltpu.VMEM((8, H), jnp.float32),
        pltpu.VMEM_SHARED((NPAD, H), jnp.float32),
        pltpu.SemaphoreType.DMA,
    ],
)(_sc_group_body)


def _sc_final_body(hs_r, rank_r, rc0_r, rc1_r, hout_r, rca_r, rcb_r,
                   idx80, rows80, idx128, rows128, sem):
    cid = lax.axis_index("c")
    sid = lax.axis_index("s")
    w = sid * 2 + cid

    def hb(t, carry):
        j = pl.multiple_of((w + t * NW) * 80, 8)
        pltpu.sync_copy(rank_r.at[pl.ds(j, 80)], idx80)
        pltpu.async_copy(hs_r.at[idx80], rows80, sem).wait()
        pltpu.sync_copy(rows80, hout_r.at[pl.ds(j, 80)])
        return carry

    lax.fori_loop(0, (N // 80 + NW - 1 - w) // NW, hb, 0)

    wo = pl.multiple_of(w * 128, 8)
    pltpu.sync_copy(rc0_r.at[pl.ds(wo, 128)], idx128)
    pltpu.async_copy(hs_r.at[idx128], rows128, sem).wait()
    pltpu.sync_copy(rows128, rca_r.at[pl.ds(wo, 128)])
    pltpu.sync_copy(rc1_r.at[pl.ds(wo, 128)], idx128)
    pltpu.async_copy(hs_r.at[idx128], rows128, sem).wait()
    pltpu.sync_copy(rows128, rcb_r.at[pl.ds(wo, 128)])


_sc_final = functools.partial(
    pl.kernel,
    out_type=(
        jax.ShapeDtypeStruct((N, H), jnp.float32),
        jax.ShapeDtypeStruct((P, H), jnp.float32),
        jax.ShapeDtypeStruct((P, H), jnp.float32),
    ),
    mesh=plsc.VectorSubcoreMesh(core_axis_name="c", subcore_axis_name="s"),
    scratch_types=[
        pltpu.VMEM((80,), jnp.int32),
        pltpu.VMEM((80, H), jnp.float32),
        pltpu.VMEM((128,), jnp.int32),
        pltpu.VMEM((128, H), jnp.float32),
        pltpu.SemaphoreType.DMA,
    ],
)(_sc_final_body)


def _init_body(ab1, aw2, ab2, aw3, ab3, nb1, nw2, nb2, nw3, nb3,
               hs_o, ma_o, mn_o):
    def c_row(b1, w2, b2, w3, b3):
        x1 = jnp.broadcast_to(jax.nn.relu(b1[...]), (8, M))
        x2 = jax.nn.relu(
            jnp.dot(x1, w2[...], preferred_element_type=jnp.float32) + b2[...])
        return jnp.dot(x2, w3[...], preferred_element_type=jnp.float32) + b3[...]

    ca = c_row(ab1, aw2, ab2, aw3, ab3)
    cn = c_row(nb1, nw2, nb2, nw3, nb3)
    hs_o[...] = jnp.zeros((256, H), jnp.float32)
    ma_o[...] = jnp.broadcast_to(ca[0:1, :], (256, H))
    mn_o[...] = jnp.broadcast_to(cn[0:1, :], (256, H))


def _tc_init(ab1, aw2, ab2, aw3, ab3, nb1, nw2, nb2, nw3, nb3):
    vspec = pl.BlockSpec(memory_space=pltpu.VMEM)
    ospec = pl.BlockSpec((256, H), lambda i: (i, 0))
    oshape = jax.ShapeDtypeStruct((NPAD, H), jnp.float32)
    return pl.pallas_call(
        _init_body,
        grid=(NPAD // 256,),
        in_specs=[vspec] * 10,
        out_specs=[ospec] * 3,
        out_shape=[oshape] * 3,
    )(ab1, aw2, ab2, aw3, ab3, nb1, nw2, nb2, nw3, nb3)


def _group_body(meta, agg, hs_i, ma_i, mn_i,
                wih, bih, bhh,
                aw1, ab1, aw2, ab2, aw3, ab3,
                nw1, nb1, nw2, nb2, nw3, nb3,
                hs_o, ma_o, mn_o,
                abuf, hbuf, mabuf, mnbuf, sem):
    del hs_i, ma_i, mn_i
    base = meta[2]
    nbig = meta[4]
    ntail = meta[5]

    def mlp3(x, w1, b1, w2, b2, w3, b3):
        x = jax.nn.relu(
            jnp.dot(x, w1[...], preferred_element_type=jnp.float32) + b1[...])
        x = jax.nn.relu(
            jnp.dot(x, w2[...], preferred_element_type=jnp.float32) + b2[...])
        return jnp.dot(x, w3[...], preferred_element_type=jnp.float32) + b3[...]

    def tile(off, tr):
        d0 = pltpu.make_async_copy(agg.at[0, pl.ds(off, tr), :],
                                   abuf.at[0, 0:tr], sem)
        d1 = pltpu.make_async_copy(agg.at[1, pl.ds(off, tr), :],
                                   abuf.at[1, 0:tr], sem)
        d0.start()
        d1.start()
        d0.wait()
        d1.wait()
        a = abuf[0, 0:tr] + abuf[1, 0:tr]
        gi = jnp.dot(a, wih[...], preferred_element_type=jnp.float32) + bih[...]
        bh = bhh[...]
        r = jax.nn.sigmoid(gi[:, :H] + bh[:, :H])
        z = jax.nn.sigmoid(gi[:, H:2 * H] + bh[:, H:2 * H])
        nn = jnp.tanh(gi[:, 2 * H:] + r * bh[:, 2 * H:])
        h = (1.0 - z) * nn
        hbuf[0:tr] = h
        mabuf[0:tr] = mlp3(h, aw1, ab1, aw2, ab2, aw3, ab3)
        mnbuf[0:tr] = mlp3(h, nw1, nb1, nw2, nb2, nw3, nb3)
        o0 = pltpu.make_async_copy(hbuf.at[0:tr], hs_o.at[pl.ds(off, tr)], sem)
        o1 = pltpu.make_async_copy(mabuf.at[0:tr], ma_o.at[pl.ds(off, tr)], sem)
        o2 = pltpu.make_async_copy(mnbuf.at[0:tr], mn_o.at[pl.ds(off, tr)], sem)
        o0.start()
        o1.start()
        o2.start()
        o0.wait()
        o1.wait()
        o2.wait()

    def big(t, carry):
        tile(base + t * 256, 256)
        return carry

    lax.fori_loop(0, nbig, big, 0)
    tail_base = base + nbig * 256

    def small(t, carry):
        tile(tail_base + t * 8, 8)
        return carry

    lax.fori_loop(0, ntail, small, 0)


def _tc_group(meta, agg, hs, ma, mn, wih, bih, bhh, aw, nw):
    aspec = pl.BlockSpec(memory_space=pl.ANY)
    vspec = pl.BlockSpec(memory_space=pltpu.VMEM)
    sspec = pl.BlockSpec(memory_space=pltpu.SMEM)
    oshape = jax.ShapeDtypeStruct((NPAD, H), jnp.float32)
    return pl.pallas_call(
        _group_body,
        in_specs=[sspec, aspec, aspec, aspec, aspec] + [vspec] * 15,
        out_specs=[aspec] * 3,
        out_shape=[oshape] * 3,
        input_output_aliases={2: 0, 3: 1, 4: 2},
        scratch_shapes=[
            pltpu.VMEM((2, 256, H), jnp.float32),
            pltpu.VMEM((256, H), jnp.float32),
            pltpu.VMEM((256, H), jnp.float32),
            pltpu.VMEM((256, H), jnp.float32),
            pltpu.SemaphoreType.DMA,
        ],
    )(meta, agg, hs, ma, mn, wih, bih, bhh, *aw, *nw)


def _prob_body(h, w1, b1, g1, e1, w2, b2, g2, e2, w3t, b3, out):
    y1 = jnp.dot(h[...], w1[...], preferred_element_type=jnp.float32) + b1[...]
    x1 = jax.nn.relu(y1 * (g1[...] * BN_S) + e1[...])
    y2 = jnp.dot(x1, w2[...], preferred_element_type=jnp.float32) + b2[...]
    x2 = jax.nn.relu(y2 * (g2[...] * BN_S) + e2[...])
    p = jnp.sum(x2 * w3t[...], axis=1, keepdims=True) + b3[...]
    out[...] = jnp.broadcast_to(p, out.shape)


def _tc_prob(h, w1, b1, g1, e1, w2, b2, g2, e2, w3t, b3):
    vspec = pl.BlockSpec(memory_space=pltpu.VMEM)
    return pl.pallas_call(
        _prob_body,
        grid=(25,),
        in_specs=[pl.BlockSpec((400, H), lambda i: (i, 0))] + [vspec] * 10,
        out_specs=pl.BlockSpec((400, H), lambda i: (i, 0)),
        out_shape=jax.ShapeDtypeStruct((N, H), jnp.float32),
    )(h, w1, b1, g1, e1, w2, b2, g2, e2, w3t, b3)


def _rc_body(ha, hb, w1, b1, g1, e1, w2, b2, g2, e2, w3t, b3, out):
    y1 = (jnp.dot(ha[...], w1[0:H], preferred_element_type=jnp.float32)
          + jnp.dot(hb[...], w1[H:2 * H], preferred_element_type=jnp.float32)
          + b1[...])
    x1 = jax.nn.relu(y1 * (g1[...] * BN_S) + e1[...])
    y2 = jnp.dot(x1, w2[...], preferred_element_type=jnp.float32) + b2[...]
    x2 = jax.nn.relu(y2 * (g2[...] * BN_S) + e2[...])
    p = jax.nn.sigmoid(jnp.sum(x2 * w3t[...], axis=1, keepdims=True) + b3[...])
    out[...] = jnp.broadcast_to(p, out.shape)


def _tc_rc(ha, hb, w1, b1, g1, e1, w2, b2, g2, e2, w3t, b3):
    vspec = pl.BlockSpec(memory_space=pltpu.VMEM)
    bspec = pl.BlockSpec((512, H), lambda i: (i, 0))
    return pl.pallas_call(
        _rc_body,
        grid=(P // 512,),
        in_specs=[bspec, bspec] + [vspec] * 10,
        out_specs=bspec,
        out_shape=jax.ShapeDtypeStruct((P, H), jnp.float32),
    )(ha, hb, w1, b1, g1, e1, w2, b2, g2, e2, w3t, b3)


def kernel(x, edge_index, forward_level, backward_level, gate, forward_index,
           rc_pair_index, params):
    del x, backward_level, forward_index
    p = params
    fl = forward_level.astype(jnp.int32)
    g = gate[:, 0].astype(jnp.int32)
    src = edge_index[0].astype(jnp.int32)
    dst = edge_index[1].astype(jnp.int32)

    upd = (fl >= 1) & (g >= 1)
    gid = jnp.where(upd, (fl - 1) * 2 + (g - 1), NGROUP).astype(jnp.int32)

    counts = jnp.bincount(gid, length=NGROUP + 1).astype(jnp.int32)
    pad8 = ((counts + 7) // 8) * 8
    pad8 = pad8.at[NGROUP].set(counts[NGROUP])
    z1 = jnp.zeros((1,), jnp.int32)
    ns = jnp.concatenate([z1, jnp.cumsum(pad8).astype(jnp.int32)])
    us = jnp.concatenate([z1, jnp.cumsum(counts).astype(jnp.int32)])
    nperm = jnp.argsort(gid, stable=True).astype(jnp.int32)
    gsort = gid[nperm]
    posn = ns[gsort] + (jnp.arange(N, dtype=jnp.int32) - us[gsort])
    rank = jnp.zeros((N,), jnp.int32).at[nperm].set(posn)

    egid = jnp.where(upd[dst], gid[dst], NGROUP).astype(jnp.int32)
    ecounts = jnp.bincount(egid, length=NGROUP + 1).astype(jnp.int32)
    epad = ((ecounts + CH - 1) // CH) * CH
    epad = epad.at[NGROUP].set(ecounts[NGROUP])
    es = jnp.concatenate([z1, jnp.cumsum(epad).astype(jnp.int32)])
    eus = jnp.concatenate([z1, jnp.cumsum(ecounts).astype(jnp.int32)])
    eperm = jnp.argsort(egid, stable=True).astype(jnp.int32)
    egsort = egid[eperm]
    epos = es[egsort] + (jnp.arange(E, dtype=jnp.int32) - eus[egsort])
    srs = jnp.zeros((EPAD,), jnp.int32).at[epos].set(rank[src[eperm]])
    drs = jnp.full((EPAD,), TRASH, jnp.int32).at[epos].set(rank[dst[eperm]])

    ks = jnp.arange(NGROUP)
    metas = jnp.stack([
        es[ks], epad[ks] // CH, ns[ks], pad8[ks] // 8,
        pad8[ks] // 256, (pad8[ks] % 256) // 8,
    ], axis=1).astype(jnp.int32)
    metas = jnp.pad(metas, ((0, 0), (0, 16 - metas.shape[1])))

    rc0r = rank[rc_pair_index[0].astype(jnp.int32)]
    rc1r = rank[rc_pair_index[1].astype(jnp.int32)]

    def row(v):
        return v.reshape(1, -1)

    aw = (p['and_w1'], row(p['and_b1']), p['and_w2'], row(p['and_b2']),
          p['and_w3'], row(p['and_b3']))
    nw = (p['not_w1'], row(p['not_b1']), p['not_w2'], row(p['not_b2']),
          p['not_w3'], row(p['not_b3']))

    hs, ma, mn = _tc_init(aw[1], aw[2], aw[3], aw[4], aw[5],
                          nw[1], nw[2], nw[3], nw[4], nw[5])

    for k in range(NGROUP):
        pre = 'and' if k % 2 == 0 else 'not'
        msg_t = ma if pre == 'and' else mn
        agg = _sc_group(msg_t, srs, drs, metas[k])
        hs, ma, mn = _tc_group(
            metas[k], agg, hs, ma, mn,
            p[pre + '_wih'], row(p[pre + '_bih']), row(p[pre + '_bhh']),
            aw, nw)

    h_out, rca, rcb = _sc_final(hs, rank, rc0r, rc1r)

    prob_full = _tc_prob(
        h_out, p['prob_w1'], row(p['prob_b1']), row(p['prob_g1']),
        row(p['prob_be1']), p['prob_w2'], row(p['prob_b2']),
        row(p['prob_g2']), row(p['prob_be2']),
        p['prob_w3'].reshape(1, -1), p['prob_b3'].reshape(1, 1))
    rc_full = _tc_rc(
        rca, rcb, p['rc_w1'], row(p['rc_b1']), row(p['rc_g1']),
        row(p['rc_be1']), p['rc_w2'], row(p['rc_b2']),
        row(p['rc_g2']), row(p['rc_be2']),
        p['rc_w3'].reshape(1, -1), p['rc_b3'].reshape(1, 1))

    return (h_out, prob_full[:, 0:1], rc_full[:, 0:1])


# speedup vs baseline: 11.3603x; 4.7671x over previous
import functools

import jax
import jax.numpy as jnp
from jax import lax
from jax.experimental import pallas as pl
from jax.experimental.pallas import tpu as pltpu
from jax.experimental.pallas import tpu_sc as plsc

N = 10000
E = 160000
P = 4096
H = 128
M = 512
NPAD = 12288
TRASH = NPAD - 1
CH = 128
EPAD = E + 14 * CH
EPADS = EPAD + 128
NGROUP = 14
NW = 32
CHA = 1280
CHB = 1024
EE = E + 14 * CH + CHB
BN_S = float(1.0 / (1.0 + 1e-5) ** 0.5)


def _iota16():
    return lax.iota(jnp.int32, 16)


def _sc_scalar(vec, j):
    return jnp.sum(jnp.where(_iota16() == j, vec, 0))


def _sc_ranks_body(rank_r, src_r, dst_r, rc0_r, rc1_r,
                   rs_o, rd_o, r0_o, r1_o,
                   rank_v, cbuf, obuf, cbuf1, obuf1, sem):
    del sem
    cid = lax.axis_index("c")
    sid = lax.axis_index("s")
    w = sid * 2 + cid
    pltpu.sync_copy(rank_r, rank_v)

    def gath(in_hbm, out_hbm, t, ib, ob, chp):
        off = pl.multiple_of(t * chp, 8)
        pltpu.sync_copy(in_hbm.at[pl.ds(off, chp)], ib)

        def it(i, carry):
            o = pl.multiple_of(i * 16, 8)
            idx = ib[pl.ds(o, 16)]
            ob[pl.ds(o, 16)] = plsc.load_gather(rank_v, [idx])
            return carry

        lax.fori_loop(0, chp // 16, it, 0)
        pltpu.sync_copy(ob, out_hbm.at[pl.ds(off, chp)])

    def body(q, carry):
        t = w + q * NW
        gath(src_r, rs_o, t, cbuf, obuf, CHA)
        gath(dst_r, rd_o, t, cbuf, obuf, CHA)
        return carry

    lax.fori_loop(0, (E // CHA + NW - 1 - w) // NW, body, 0)
    gath(rc0_r, r0_o, w, cbuf1, obuf1, 128)
    gath(rc1_r, r1_o, w, cbuf1, obuf1, 128)


_sc_ranks = functools.partial(
    pl.kernel,
    out_type=(
        jax.ShapeDtypeStruct((E,), jnp.int32),
        jax.ShapeDtypeStruct((E,), jnp.int32),
        jax.ShapeDtypeStruct((P,), jnp.int32),
        jax.ShapeDtypeStruct((P,), jnp.int32),
    ),
    mesh=plsc.VectorSubcoreMesh(core_axis_name="c", subcore_axis_name="s"),
    compiler_params=pltpu.CompilerParams(needs_layout_passes=False),
    scratch_types=[
        pltpu.VMEM((N,), jnp.int32),
        pltpu.VMEM((CHA,), jnp.int32),
        pltpu.VMEM((CHA,), jnp.int32),
        pltpu.VMEM((128,), jnp.int32),
        pltpu.VMEM((128,), jnp.int32),
        pltpu.SemaphoreType.DMA,
    ],
)(_sc_ranks_body)


def _sc_slots_body(rs_r, rd_r, pos_r, srs_o, drs_o,
                   vbuf, pbuf, sem):
    cid = lax.axis_index("c")
    sid = lax.axis_index("s")
    w = sid * 2 + cid

    def body(q, carry):
        off = pl.multiple_of((w + q * NW) * CHB, 8)
        pltpu.sync_copy(pos_r.at[pl.ds(off, CHB)], pbuf)
        pltpu.sync_copy(rs_r.at[pl.ds(off, CHB)], vbuf)
        pltpu.async_copy(vbuf, srs_o.at[pbuf], sem).wait()
        pltpu.sync_copy(rd_r.at[pl.ds(off, CHB)], vbuf)
        pltpu.async_copy(vbuf, drs_o.at[pbuf], sem).wait()
        return carry

    lax.fori_loop(0, (EE // CHB + NW - 1 - w) // NW, body, 0)


_sc_slots = functools.partial(
    pl.kernel,
    out_type=(
        jax.ShapeDtypeStruct((EPADS,), jnp.int32),
        jax.ShapeDtypeStruct((EPADS,), jnp.int32),
    ),
    mesh=plsc.VectorSubcoreMesh(core_axis_name="c", subcore_axis_name="s"),
    compiler_params=pltpu.CompilerParams(needs_layout_passes=False),
    scratch_types=[
        pltpu.VMEM((CHB,), jnp.int32),
        pltpu.VMEM((CHB,), jnp.int32),
        pltpu.SemaphoreType.DMA,
    ],
)(_sc_slots_body)


def _sc_group_body(msg_t, srs_r, drs_r, meta_r, agg_out,
                   meta_v, sidx, didx, rows, zbuf, aggsh, sem):
    cid = lax.axis_index("c")
    sid = lax.axis_index("s")
    w = sid * 2 + cid
    pltpu.sync_copy(meta_r, meta_v)
    mv = meta_v[...]
    es_k = _sc_scalar(mv, 0)
    nch = _sc_scalar(mv, 1)
    ns_k = _sc_scalar(mv, 2)
    noct = _sc_scalar(mv, 3)
    for r in range(8):
        for c in range(8):
            zbuf[r, pl.ds(c * 16, 16)] = jnp.zeros((16,), jnp.float32)

    def zbody(t, carry):
        j = pl.multiple_of(ns_k + (sid + t * 16) * 8, 8)
        pltpu.sync_copy(zbuf, aggsh.at[pl.ds(j, 8)])
        return carry

    lax.fori_loop(0, (noct + 15 - sid) // 16, zbody, 0)
    plsc.subcore_barrier()

    def ebody(t, carry):
        off = pl.multiple_of(es_k + (w + t * NW) * CH, 8)
        pltpu.sync_copy(srs_r.at[pl.ds(off, CH)], sidx)
        pltpu.sync_copy(drs_r.at[pl.ds(off, CH)], didx)
        pltpu.async_copy(msg_t.at[sidx], rows, sem).wait()
        pltpu.sync_copy(rows, aggsh.at[didx], add=True)
        return carry

    lax.fori_loop(0, (nch + NW - 1 - w) // NW, ebody, 0)
    plsc.subcore_barrier()

    def obody(t, carry):
        j = pl.multiple_of(ns_k + (sid + t * 16) * 8, 8)
        pltpu.sync_copy(aggsh.at[pl.ds(j, 8)],
                        agg_out.at[cid, pl.ds(j, 8)])
        return carry

    lax.fori_loop(0, (noct + 15 - sid) // 16, obody, 0)


_sc_group = functools.partial(
    pl.kernel,
    out_type=jax.ShapeDtypeStruct((2, NPAD, H), jnp.float32),
    mesh=plsc.VectorSubcoreMesh(core_axis_name="c", subcore_axis_name="s"),
    compiler_params=pltpu.CompilerParams(needs_layout_passes=False),
    scratch_types=[
        pltpu.VMEM((16,), jnp.int32),
        pltpu.VMEM((CH,), jnp.int32),
        pltpu.VMEM((CH,), jnp.int32),
        pltpu.VMEM((CH, H), jnp.float32),
        pltpu.VMEM((8, H), jnp.float32),
        pltpu.VMEM_SHARED((NPAD, H), jnp.float32),
        pltpu.SemaphoreType.DMA,
    ],
)(_sc_group_body)


def _sc_final_body(hs_r, rank_r, rc0_r, rc1_r, hout_r, rca_r, rcb_r,
                   idx80, rows80, idx128, rows128, sem):
    cid = lax.axis_index("c")
    sid = lax.axis_index("s")
    w = sid * 2 + cid

    def hb(t, carry):
        j = pl.multiple_of((w + t * NW) * 80, 8)
        pltpu.sync_copy(rank_r.at[pl.ds(j, 80)], idx80)
        pltpu.async_copy(hs_r.at[idx80], rows80, sem).wait()
        pltpu.sync_copy(rows80, hout_r.at[pl.ds(j, 80)])
        return carry

    lax.fori_loop(0, (N // 80 + NW - 1 - w) // NW, hb, 0)

    wo = pl.multiple_of(w * 128, 8)
    pltpu.sync_copy(rc0_r.at[pl.ds(wo, 128)], idx128)
    pltpu.async_copy(hs_r.at[idx128], rows128, sem).wait()
    pltpu.sync_copy(rows128, rca_r.at[pl.ds(wo, 128)])
    pltpu.sync_copy(rc1_r.at[pl.ds(wo, 128)], idx128)
    pltpu.async_copy(hs_r.at[idx128], rows128, sem).wait()
    pltpu.sync_copy(rows128, rcb_r.at[pl.ds(wo, 128)])


_sc_final = functools.partial(
    pl.kernel,
    out_type=(
        jax.ShapeDtypeStruct((N, H), jnp.float32),
        jax.ShapeDtypeStruct((P, H), jnp.float32),
        jax.ShapeDtypeStruct((P, H), jnp.float32),
    ),
    mesh=plsc.VectorSubcoreMesh(core_axis_name="c", subcore_axis_name="s"),
    compiler_params=pltpu.CompilerParams(needs_layout_passes=False),
    scratch_types=[
        pltpu.VMEM((80,), jnp.int32),
        pltpu.VMEM((80, H), jnp.float32),
        pltpu.VMEM((128,), jnp.int32),
        pltpu.VMEM((128, H), jnp.float32),
        pltpu.SemaphoreType.DMA,
    ],
)(_sc_final_body)


def _init_body(ab1, aw2, ab2, aw3, ab3, nb1, nw2, nb2, nw3, nb3,
               hs_o, ma_o, mn_o):
    def c_row(b1, w2, b2, w3, b3):
        x1 = jnp.broadcast_to(jax.nn.relu(b1[...]), (8, M))
        x2 = jax.nn.relu(
            jnp.dot(x1, w2[...], preferred_element_type=jnp.float32) + b2[...])
        return jnp.dot(x2, w3[...], preferred_element_type=jnp.float32) + b3[...]

    ca = c_row(ab1, aw2, ab2, aw3, ab3)
    cn = c_row(nb1, nw2, nb2, nw3, nb3)
    hs_o[...] = jnp.zeros((256, H), jnp.float32)
    ma_o[...] = jnp.broadcast_to(ca[0:1, :], (256, H))
    mn_o[...] = jnp.broadcast_to(cn[0:1, :], (256, H))


def _tc_init(ab1, aw2, ab2, aw3, ab3, nb1, nw2, nb2, nw3, nb3):
    vspec = pl.BlockSpec(memory_space=pltpu.VMEM)
    ospec = pl.BlockSpec((256, H), lambda i: (i, 0))
    oshape = jax.ShapeDtypeStruct((NPAD, H), jnp.float32)
    return pl.pallas_call(
        _init_body,
        grid=(NPAD // 256,),
        in_specs=[vspec] * 10,
        out_specs=[ospec] * 3,
        out_shape=[oshape] * 3,
    )(ab1, aw2, ab2, aw3, ab3, nb1, nw2, nb2, nw3, nb3)


def _group_body(meta, agg, hs_i, ma_i, mn_i,
                wih, bih, bhh,
                aw1, ab1, aw2, ab2, aw3, ab3,
                nw1, nb1, nw2, nb2, nw3, nb3,
                hs_o, ma_o, mn_o,
                abuf, hbuf, mabuf, mnbuf, sem):
    del hs_i, ma_i, mn_i
    base = meta[2]
    ntile = meta[4]

    def mlp3(x, w1, b1, w2, b2, w3, b3):
        x = jax.nn.relu(
            jnp.dot(x, w1[...], preferred_element_type=jnp.float32) + b1[...])
        x = jax.nn.relu(
            jnp.dot(x, w2[...], preferred_element_type=jnp.float32) + b2[...])
        return jnp.dot(x, w3[...], preferred_element_type=jnp.float32) + b3[...]

    def tile(t, carry):
        off = base + t * 128
        d0 = pltpu.make_async_copy(agg.at[0, pl.ds(off, 128), :],
                                   abuf.at[0], sem)
        d1 = pltpu.make_async_copy(agg.at[1, pl.ds(off, 128), :],
                                   abuf.at[1], sem)
        d0.start()
        d1.start()
        d0.wait()
        d1.wait()
        a = abuf[0] + abuf[1]
        gi = jnp.dot(a, wih[...], preferred_element_type=jnp.float32) + bih[...]
        bh = bhh[...]
        r = jax.nn.sigmoid(gi[:, :H] + bh[:, :H])
        z = jax.nn.sigmoid(gi[:, H:2 * H] + bh[:, H:2 * H])
        nn = jnp.tanh(gi[:, 2 * H:] + r * bh[:, 2 * H:])
        h = (1.0 - z) * nn
        hbuf[...] = h
        mabuf[...] = mlp3(h, aw1, ab1, aw2, ab2, aw3, ab3)
        mnbuf[...] = mlp3(h, nw1, nb1, nw2, nb2, nw3, nb3)
        o0 = pltpu.make_async_copy(hbuf, hs_o.at[pl.ds(off, 128)], sem)
        o1 = pltpu.make_async_copy(mabuf, ma_o.at[pl.ds(off, 128)], sem)
        o2 = pltpu.make_async_copy(mnbuf, mn_o.at[pl.ds(off, 128)], sem)
        o0.start()
        o1.start()
        o2.start()
        o0.wait()
        o1.wait()
        o2.wait()
        return carry

    lax.fori_loop(0, ntile, tile, 0)


def _tc_group(meta, agg, hs, ma, mn, wih, bih, bhh, aw, nw):
    aspec = pl.BlockSpec(memory_space=pl.ANY)
    vspec = pl.BlockSpec(memory_space=pltpu.VMEM)
    sspec = pl.BlockSpec(memory_space=pltpu.SMEM)
    oshape = jax.ShapeDtypeStruct((NPAD, H), jnp.float32)
    return pl.pallas_call(
        _group_body,
        in_specs=[sspec, aspec, aspec, aspec, aspec] + [vspec] * 15,
        out_specs=[aspec] * 3,
        out_shape=[oshape] * 3,
        input_output_aliases={2: 0, 3: 1, 4: 2},
        scratch_shapes=[
            pltpu.VMEM((2, 128, H), jnp.float32),
            pltpu.VMEM((128, H), jnp.float32),
            pltpu.VMEM((128, H), jnp.float32),
            pltpu.VMEM((128, H), jnp.float32),
            pltpu.SemaphoreType.DMA,
        ],
    )(meta, agg, hs, ma, mn, wih, bih, bhh, *aw, *nw)


def _prob_body(h, w1, b1, g1, e1, w2, b2, g2, e2, w3t, b3, out):
    y1 = jnp.dot(h[...], w1[...], preferred_element_type=jnp.float32) + b1[...]
    x1 = jax.nn.relu(y1 * (g1[...] * BN_S) + e1[...])
    y2 = jnp.dot(x1, w2[...], preferred_element_type=jnp.float32) + b2[...]
    x2 = jax.nn.relu(y2 * (g2[...] * BN_S) + e2[...])
    p = jnp.sum(x2 * w3t[...], axis=1, keepdims=True) + b3[...]
    out[...] = jnp.broadcast_to(p, out.shape)


def _tc_prob(h, w1, b1, g1, e1, w2, b2, g2, e2, w3t, b3):
    vspec = pl.BlockSpec(memory_space=pltpu.VMEM)
    return pl.pallas_call(
        _prob_body,
        grid=(25,),
        in_specs=[pl.BlockSpec((400, H), lambda i: (i, 0))] + [vspec] * 10,
        out_specs=pl.BlockSpec((400, H), lambda i: (i, 0)),
        out_shape=jax.ShapeDtypeStruct((N, H), jnp.float32),
    )(h, w1, b1, g1, e1, w2, b2, g2, e2, w3t, b3)


def _rc_body(ha, hb, w1, b1, g1, e1, w2, b2, g2, e2, w3t, b3, out):
    y1 = (jnp.dot(ha[...], w1[0:H], preferred_element_type=jnp.float32)
          + jnp.dot(hb[...], w1[H:2 * H], preferred_element_type=jnp.float32)
          + b1[...])
    x1 = jax.nn.relu(y1 * (g1[...] * BN_S) + e1[...])
    y2 = jnp.dot(x1, w2[...], preferred_element_type=jnp.float32) + b2[...]
    x2 = jax.nn.relu(y2 * (g2[...] * BN_S) + e2[...])
    p = jax.nn.sigmoid(jnp.sum(x2 * w3t[...], axis=1, keepdims=True) + b3[...])
    out[...] = jnp.broadcast_to(p, out.shape)


def _tc_rc(ha, hb, w1, b1, g1, e1, w2, b2, g2, e2, w3t, b3):
    vspec = pl.BlockSpec(memory_space=pltpu.VMEM)
    bspec = pl.BlockSpec((512, H), lambda i: (i, 0))
    return pl.pallas_call(
        _rc_body,
        grid=(P // 512,),
        in_specs=[bspec, bspec] + [vspec] * 10,
        out_specs=bspec,
        out_shape=jax.ShapeDtypeStruct((P, H), jnp.float32),
    )(ha, hb, w1, b1, g1, e1, w2, b2, g2, e2, w3t, b3)


def kernel(x, edge_index, forward_level, backward_level, gate, forward_index,
           rc_pair_index, params):
    del x, backward_level, forward_index
    p = params
    fl = forward_level.astype(jnp.int32)
    g = gate[:, 0].astype(jnp.int32)
    src = edge_index[0].astype(jnp.int32)
    dst = edge_index[1].astype(jnp.int32)

    upd = (fl >= 1) & (g >= 1)
    gid = jnp.where(upd, (fl - 1) * 2 + (g - 1), NGROUP).astype(jnp.int32)
    ks = jnp.arange(NGROUP + 1, dtype=jnp.int32)
    onehot_n = gid[None, :] == ks[:, None]
    counts = jnp.sum(onehot_n, axis=1).astype(jnp.int32)
    occ_n = jnp.cumsum(onehot_n.astype(jnp.int32), axis=1)
    pad128 = ((counts[:NGROUP] + 127) // 128) * 128
    z1 = jnp.zeros((1,), jnp.int32)
    ns = jnp.concatenate([z1, jnp.cumsum(pad128).astype(jnp.int32)])
    rank = jnp.sum(
        jnp.where(onehot_n, ns[:, None] + occ_n - 1, 0), axis=0
    ).astype(jnp.int32)

    rc0 = rc_pair_index[0].astype(jnp.int32)
    rc1 = rc_pair_index[1].astype(jnp.int32)
    rs_e, rd_e, rc0r, rc1r = _sc_ranks(rank, src, dst, rc0, rc1)

    egid = jnp.sum((rd_e[None, :] >= ns[1:, None]).astype(jnp.int32),
                   axis=0).astype(jnp.int32)
    onehot_e = egid[None, :] == ks[:, None]
    ecounts = jnp.sum(onehot_e, axis=1).astype(jnp.int32)
    occ_e = jnp.cumsum(onehot_e.astype(jnp.int32), axis=1)
    epad = ((ecounts[:NGROUP] + CH - 1) // CH) * CH
    es = jnp.concatenate([z1, jnp.cumsum(epad).astype(jnp.int32)])
    epos = jnp.sum(
        jnp.where(onehot_e, es[:, None] + occ_e - 1, 0), axis=0
    ).astype(jnp.int32)

    jj = jnp.arange(CH, dtype=jnp.int32)[None, :]
    ec14 = ecounts[:NGROUP]
    pp = jnp.where(jj < (epad - ec14)[:, None],
                   (es[:NGROUP] + ec14)[:, None] + jj,
                   EPAD + jj).astype(jnp.int32).ravel()
    sink = (EPAD + (jnp.arange(CHB, dtype=jnp.int32) % 128)).astype(jnp.int32)
    npad_e = NGROUP * CH
    rs_ext = jnp.concatenate([rs_e, jnp.zeros((npad_e + CHB,), jnp.int32)])
    rd_ext = jnp.concatenate([rd_e, jnp.full((npad_e + CHB,), TRASH, jnp.int32)])
    pos_ext = jnp.concatenate([epos, pp, sink])

    srs, drs = _sc_slots(rs_ext, rd_ext, pos_ext)

    metas = jnp.stack([
        es[:NGROUP], epad // CH, ns[:NGROUP], pad128 // 8, pad128 // 128,
    ], axis=1).astype(jnp.int32)
    metas = jnp.pad(metas, ((0, 0), (0, 16 - metas.shape[1])))

    def row(v):
        return v.reshape(1, -1)

    aw = (p['and_w1'], row(p['and_b1']), p['and_w2'], row(p['and_b2']),
          p['and_w3'], row(p['and_b3']))
    nw = (p['not_w1'], row(p['not_b1']), p['not_w2'], row(p['not_b2']),
          p['not_w3'], row(p['not_b3']))

    hs, ma, mn = _tc_init(aw[1], aw[2], aw[3], aw[4], aw[5],
                          nw[1], nw[2], nw[3], nw[4], nw[5])

    for k in range(NGROUP):
        pre = 'and' if k % 2 == 0 else 'not'
        msg_t = ma if pre == 'and' else mn
        agg = _sc_group(msg_t, srs, drs, metas[k])
        hs, ma, mn = _tc_group(
            metas[k], agg, hs, ma, mn,
            p[pre + '_wih'], row(p[pre + '_bih']), row(p[pre + '_bhh']),
            aw, nw)

    h_out, rca, rcb = _sc_final(hs, rank, rc0r, rc1r)

    prob_full = _tc_prob(
        h_out, p['prob_w1'], row(p['prob_b1']), row(p['prob_g1']),
        row(p['prob_be1']), p['prob_w2'], row(p['prob_b2']),
        row(p['prob_g2']), row(p['prob_be2']),
        p['prob_w3'].reshape(1, -1), p['prob_b3'].reshape(1, 1))
    rc_full = _tc_rc(
        rca, rcb, p['rc_w1'], row(p['rc_b1']), row(p['rc_g1']),
        row(p['rc_be1']), p['rc_w2'], row(p['rc_b2']),
        row(p['rc_g2']), row(p['rc_be2']),
        p['rc_w3'].reshape(1, -1), p['rc_b3'].reshape(1, 1))

    return (h_out, prob_full[:, 0:1], rc_full[:, 0:1])


# speedup vs baseline: 29.3925x; 2.5873x over previous
import functools

import jax
import jax.numpy as jnp
from jax import lax
from jax.experimental import pallas as pl
from jax.experimental.pallas import tpu as pltpu
from jax.experimental.pallas import tpu_sc as plsc

N = 10000
E = 160000
P = 4096
H = 128
M = 512
NPAD = 12288
TRASH = NPAD - 1
CH = 128
EPAD = E + 14 * CH
EPADS = 163840
NGROUP = 14
NW = 32
CHA = 1280
CHB = 1024
EE = E + 14 * CH + CHB
BN_S = float(1.0 / (1.0 + 1e-5) ** 0.5)


def _iota16():
    return lax.iota(jnp.int32, 16)


def _sc_scalar(vec, j):
    return jnp.sum(jnp.where(_iota16() == j, vec, 0))


def _sc_ranks_body(rank_r, src_r, dst_r, rc0_r, rc1_r,
                   rs_o, rd_o, r0_o, r1_o,
                   rank_v, cbuf, obuf, cbuf1, obuf1, sem):
    del sem
    cid = lax.axis_index("c")
    sid = lax.axis_index("s")
    w = sid * 2 + cid
    pltpu.sync_copy(rank_r, rank_v)

    def gath(in_hbm, out_hbm, t, ib, ob, chp):
        off = pl.multiple_of(t * chp, 8)
        pltpu.sync_copy(in_hbm.at[pl.ds(off, chp)], ib)

        def it(i, carry):
            o = pl.multiple_of(i * 16, 8)
            idx = ib[pl.ds(o, 16)]
            ob[pl.ds(o, 16)] = plsc.load_gather(rank_v, [idx])
            return carry

        lax.fori_loop(0, chp // 16, it, 0)
        pltpu.sync_copy(ob, out_hbm.at[pl.ds(off, chp)])

    def body(q, carry):
        t = w + q * NW
        gath(src_r, rs_o, t, cbuf, obuf, CHA)
        gath(dst_r, rd_o, t, cbuf, obuf, CHA)
        return carry

    lax.fori_loop(0, (E // CHA + NW - 1 - w) // NW, body, 0)
    gath(rc0_r, r0_o, w, cbuf1, obuf1, 128)
    gath(rc1_r, r1_o, w, cbuf1, obuf1, 128)


_sc_ranks = functools.partial(
    pl.kernel,
    out_type=(
        jax.ShapeDtypeStruct((E,), jnp.int32),
        jax.ShapeDtypeStruct((E,), jnp.int32),
        jax.ShapeDtypeStruct((P,), jnp.int32),
        jax.ShapeDtypeStruct((P,), jnp.int32),
    ),
    mesh=plsc.VectorSubcoreMesh(core_axis_name="c", subcore_axis_name="s"),
    compiler_params=pltpu.CompilerParams(needs_layout_passes=False),
    scratch_types=[
        pltpu.VMEM((N,), jnp.int32),
        pltpu.VMEM((CHA,), jnp.int32),
        pltpu.VMEM((CHA,), jnp.int32),
        pltpu.VMEM((128,), jnp.int32),
        pltpu.VMEM((128,), jnp.int32),
        pltpu.SemaphoreType.DMA,
    ],
)(_sc_ranks_body)


def _sc_slots_body(rs_r, rd_r, pos_r, srs_o, drs_o,
                   vbuf, pbuf, zbuf, sbuf, sem):
    del sem
    cid = lax.axis_index("c")
    sid = lax.axis_index("s")
    for i in range(CHB // 16):
        zbuf[pl.ds(i * 16, 16)] = jnp.zeros((16,), jnp.int32)

    def zb(t, carry):
        off = pl.multiple_of((sid + t * 16) * CHB, 8)
        pltpu.sync_copy(zbuf, sbuf.at[pl.ds(off, CHB)])
        return carry

    lax.fori_loop(0, (EPADS // CHB + 15 - sid) // 16, zb, 0)
    plsc.subcore_barrier()

    def scat(val_r):
        def body(q, carry):
            off = pl.multiple_of((sid + q * 16) * CHB, 8)
            pltpu.sync_copy(pos_r.at[pl.ds(off, CHB)], pbuf)
            pltpu.sync_copy(val_r.at[pl.ds(off, CHB)], vbuf)
            pltpu.sync_copy(vbuf, sbuf.at[pbuf], add=True)
            return carry

        lax.fori_loop(0, (EE // CHB + 15 - sid) // 16, body, 0)

    @pl.when(cid == 0)
    def _():
        scat(rs_r)

    @pl.when(cid == 1)
    def _():
        scat(rd_r)

    plsc.subcore_barrier()

    def out(o_r):
        def body(t, carry):
            off = pl.multiple_of((sid + t * 16) * CHB, 8)
            pltpu.sync_copy(sbuf.at[pl.ds(off, CHB)], o_r.at[pl.ds(off, CHB)])
            return carry

        lax.fori_loop(0, (EPADS // CHB + 15 - sid) // 16, body, 0)

    @pl.when(cid == 0)
    def _():
        out(srs_o)

    @pl.when(cid == 1)
    def _():
        out(drs_o)


_sc_slots = functools.partial(
    pl.kernel,
    out_type=(
        jax.ShapeDtypeStruct((EPADS,), jnp.int32),
        jax.ShapeDtypeStruct((EPADS,), jnp.int32),
    ),
    mesh=plsc.VectorSubcoreMesh(core_axis_name="c", subcore_axis_name="s"),
    compiler_params=pltpu.CompilerParams(needs_layout_passes=False),
    scratch_types=[
        pltpu.VMEM((CHB,), jnp.int32),
        pltpu.VMEM((CHB,), jnp.int32),
        pltpu.VMEM((CHB,), jnp.int32),
        pltpu.VMEM_SHARED((EPADS,), jnp.int32),
        pltpu.SemaphoreType.DMA,
    ],
)(_sc_slots_body)


def _sc_group_body(msg_t, srs_r, drs_r, meta_r, agg_out,
                   meta_v, sidx, didx, rows, zbuf, aggsh, sem):
    cid = lax.axis_index("c")
    sid = lax.axis_index("s")
    w = sid * 2 + cid
    pltpu.sync_copy(meta_r, meta_v)
    mv = meta_v[...]
    es_k = _sc_scalar(mv, 0)
    nch = _sc_scalar(mv, 1)
    ns_k = _sc_scalar(mv, 2)
    noct = _sc_scalar(mv, 3)
    for r in range(8):
        for c in range(8):
            zbuf[r, pl.ds(c * 16, 16)] = jnp.zeros((16,), jnp.float32)

    def zbody(t, carry):
        j = pl.multiple_of(ns_k + (sid + t * 16) * 8, 8)
        pltpu.sync_copy(zbuf, aggsh.at[pl.ds(j, 8)])
        return carry

    lax.fori_loop(0, (noct + 15 - sid) // 16, zbody, 0)
    plsc.subcore_barrier()

    def ebody(t, carry):
        off = pl.multiple_of(es_k + (w + t * NW) * CH, 8)
        pltpu.sync_copy(srs_r.at[pl.ds(off, CH)], sidx)
        pltpu.sync_copy(drs_r.at[pl.ds(off, CH)], didx)
        pltpu.async_copy(msg_t.at[sidx], rows, sem).wait()
        pltpu.sync_copy(rows, aggsh.at[didx], add=True)
        return carry

    lax.fori_loop(0, (nch + NW - 1 - w) // NW, ebody, 0)
    plsc.subcore_barrier()

    def obody(t, carry):
        j = pl.multiple_of(ns_k + (sid + t * 16) * 8, 8)
        pltpu.sync_copy(aggsh.at[pl.ds(j, 8)],
                        agg_out.at[cid, pl.ds(j, 8)])
        return carry

    lax.fori_loop(0, (noct + 15 - sid) // 16, obody, 0)


_sc_group = functools.partial(
    pl.kernel,
    out_type=jax.ShapeDtypeStruct((2, NPAD, H), jnp.float32),
    mesh=plsc.VectorSubcoreMesh(core_axis_name="c", subcore_axis_name="s"),
    compiler_params=pltpu.CompilerParams(needs_layout_passes=False),
    scratch_types=[
        pltpu.VMEM((16,), jnp.int32),
        pltpu.VMEM((CH,), jnp.int32),
        pltpu.VMEM((CH,), jnp.int32),
        pltpu.VMEM((CH, H), jnp.float32),
        pltpu.VMEM((8, H), jnp.float32),
        pltpu.VMEM_SHARED((NPAD, H), jnp.float32),
        pltpu.SemaphoreType.DMA,
    ],
)(_sc_group_body)


def _sc_final_body(hs_r, rank_r, rc0_r, rc1_r, hout_r, rca_r, rcb_r,
                   idx80, rows80, idx128, rows128, sem):
    cid = lax.axis_index("c")
    sid = lax.axis_index("s")
    w = sid * 2 + cid

    def hb(t, carry):
        j = pl.multiple_of((w + t * NW) * 80, 8)
        pltpu.sync_copy(rank_r.at[pl.ds(j, 80)], idx80)
        pltpu.async_copy(hs_r.at[idx80], rows80, sem).wait()
        pltpu.sync_copy(rows80, hout_r.at[pl.ds(j, 80)])
        return carry

    lax.fori_loop(0, (N // 80 + NW - 1 - w) // NW, hb, 0)

    wo = pl.multiple_of(w * 128, 8)
    pltpu.sync_copy(rc0_r.at[pl.ds(wo, 128)], idx128)
    pltpu.async_copy(hs_r.at[idx128], rows128, sem).wait()
    pltpu.sync_copy(rows128, rca_r.at[pl.ds(wo, 128)])
    pltpu.sync_copy(rc1_r.at[pl.ds(wo, 128)], idx128)
    pltpu.async_copy(hs_r.at[idx128], rows128, sem).wait()
    pltpu.sync_copy(rows128, rcb_r.at[pl.ds(wo, 128)])


_sc_final = functools.partial(
    pl.kernel,
    out_type=(
        jax.ShapeDtypeStruct((N, H), jnp.float32),
        jax.ShapeDtypeStruct((P, H), jnp.float32),
        jax.ShapeDtypeStruct((P, H), jnp.float32),
    ),
    mesh=plsc.VectorSubcoreMesh(core_axis_name="c", subcore_axis_name="s"),
    compiler_params=pltpu.CompilerParams(needs_layout_passes=False),
    scratch_types=[
        pltpu.VMEM((80,), jnp.int32),
        pltpu.VMEM((80, H), jnp.float32),
        pltpu.VMEM((128,), jnp.int32),
        pltpu.VMEM((128, H), jnp.float32),
        pltpu.SemaphoreType.DMA,
    ],
)(_sc_final_body)


def _init_body(ab1, aw2, ab2, aw3, ab3, nb1, nw2, nb2, nw3, nb3,
               hs_o, ma_o, mn_o):
    def c_row(b1, w2, b2, w3, b3):
        x1 = jnp.broadcast_to(jax.nn.relu(b1[...]), (8, M))
        x2 = jax.nn.relu(
            jnp.dot(x1, w2[...], preferred_element_type=jnp.float32) + b2[...])
        return jnp.dot(x2, w3[...], preferred_element_type=jnp.float32) + b3[...]

    ca = c_row(ab1, aw2, ab2, aw3, ab3)
    cn = c_row(nb1, nw2, nb2, nw3, nb3)
    hs_o[...] = jnp.zeros((256, H), jnp.float32)
    ma_o[...] = jnp.broadcast_to(ca[0:1, :], (256, H))
    mn_o[...] = jnp.broadcast_to(cn[0:1, :], (256, H))


def _tc_init(ab1, aw2, ab2, aw3, ab3, nb1, nw2, nb2, nw3, nb3):
    vspec = pl.BlockSpec(memory_space=pltpu.VMEM)
    ospec = pl.BlockSpec((256, H), lambda i: (i, 0))
    oshape = jax.ShapeDtypeStruct((NPAD, H), jnp.float32)
    return pl.pallas_call(
        _init_body,
        grid=(NPAD // 256,),
        in_specs=[vspec] * 10,
        out_specs=[ospec] * 3,
        out_shape=[oshape] * 3,
    )(ab1, aw2, ab2, aw3, ab3, nb1, nw2, nb2, nw3, nb3)


def _group_body(meta, agg, hs_i, ma_i, mn_i,
                wih, bih, bhh,
                aw1, ab1, aw2, ab2, aw3, ab3,
                nw1, nb1, nw2, nb2, nw3, nb3,
                hs_o, ma_o, mn_o,
                abuf, hbuf, mabuf, mnbuf, sem):
    del hs_i, ma_i, mn_i
    base = meta[2]
    ntile = meta[4]

    def mlp3(x, w1, b1, w2, b2, w3, b3):
        x = jax.nn.relu(
            jnp.dot(x, w1[...], preferred_element_type=jnp.float32) + b1[...])
        x = jax.nn.relu(
            jnp.dot(x, w2[...], preferred_element_type=jnp.float32) + b2[...])
        return jnp.dot(x, w3[...], preferred_element_type=jnp.float32) + b3[...]

    def tile(t, carry):
        off = base + t * 128
        d0 = pltpu.make_async_copy(agg.at[0, pl.ds(off, 128), :],
                                   abuf.at[0], sem)
        d1 = pltpu.make_async_copy(agg.at[1, pl.ds(off, 128), :],
                                   abuf.at[1], sem)
        d0.start()
        d1.start()
        d0.wait()
        d1.wait()
        a = abuf[0] + abuf[1]
        gi = jnp.dot(a, wih[...], preferred_element_type=jnp.float32) + bih[...]
        bh = bhh[...]
        r = jax.nn.sigmoid(gi[:, :H] + bh[:, :H])
        z = jax.nn.sigmoid(gi[:, H:2 * H] + bh[:, H:2 * H])
        nn = jnp.tanh(gi[:, 2 * H:] + r * bh[:, 2 * H:])
        h = (1.0 - z) * nn
        hbuf[...] = h
        mabuf[...] = mlp3(h, aw1, ab1, aw2, ab2, aw3, ab3)
        mnbuf[...] = mlp3(h, nw1, nb1, nw2, nb2, nw3, nb3)
        o0 = pltpu.make_async_copy(hbuf, hs_o.at[pl.ds(off, 128)], sem)
        o1 = pltpu.make_async_copy(mabuf, ma_o.at[pl.ds(off, 128)], sem)
        o2 = pltpu.make_async_copy(mnbuf, mn_o.at[pl.ds(off, 128)], sem)
        o0.start()
        o1.start()
        o2.start()
        o0.wait()
        o1.wait()
        o2.wait()
        return carry

    lax.fori_loop(0, ntile, tile, 0)


def _tc_group(meta, agg, hs, ma, mn, wih, bih, bhh, aw, nw):
    aspec = pl.BlockSpec(memory_space=pl.ANY)
    vspec = pl.BlockSpec(memory_space=pltpu.VMEM)
    sspec = pl.BlockSpec(memory_space=pltpu.SMEM)
    oshape = jax.ShapeDtypeStruct((NPAD, H), jnp.float32)
    return pl.pallas_call(
        _group_body,
        in_specs=[sspec, aspec, aspec, aspec, aspec] + [vspec] * 15,
        out_specs=[aspec] * 3,
        out_shape=[oshape] * 3,
        input_output_aliases={2: 0, 3: 1, 4: 2},
        scratch_shapes=[
            pltpu.VMEM((2, 128, H), jnp.float32),
            pltpu.VMEM((128, H), jnp.float32),
            pltpu.VMEM((128, H), jnp.float32),
            pltpu.VMEM((128, H), jnp.float32),
            pltpu.SemaphoreType.DMA,
        ],
    )(meta, agg, hs, ma, mn, wih, bih, bhh, *aw, *nw)


def _prob_body(h, w1, b1, g1, e1, w2, b2, g2, e2, w3t, b3, out):
    y1 = jnp.dot(h[...], w1[...], preferred_element_type=jnp.float32) + b1[...]
    x1 = jax.nn.relu(y1 * (g1[...] * BN_S) + e1[...])
    y2 = jnp.dot(x1, w2[...], preferred_element_type=jnp.float32) + b2[...]
    x2 = jax.nn.relu(y2 * (g2[...] * BN_S) + e2[...])
    p = jnp.sum(x2 * w3t[...], axis=1, keepdims=True) + b3[...]
    out[...] = jnp.broadcast_to(p, out.shape)


def _tc_prob(h, w1, b1, g1, e1, w2, b2, g2, e2, w3t, b3):
    vspec = pl.BlockSpec(memory_space=pltpu.VMEM)
    return pl.pallas_call(
        _prob_body,
        grid=(25,),
        in_specs=[pl.BlockSpec((400, H), lambda i: (i, 0))] + [vspec] * 10,
        out_specs=pl.BlockSpec((400, H), lambda i: (i, 0)),
        out_shape=jax.ShapeDtypeStruct((N, H), jnp.float32),
    )(h, w1, b1, g1, e1, w2, b2, g2, e2, w3t, b3)


def _rc_body(ha, hb, w1, b1, g1, e1, w2, b2, g2, e2, w3t, b3, out):
    y1 = (jnp.dot(ha[...], w1[0:H], preferred_element_type=jnp.float32)
          + jnp.dot(hb[...], w1[H:2 * H], preferred_element_type=jnp.float32)
          + b1[...])
    x1 = jax.nn.relu(y1 * (g1[...] * BN_S) + e1[...])
    y2 = jnp.dot(x1, w2[...], preferred_element_type=jnp.float32) + b2[...]
    x2 = jax.nn.relu(y2 * (g2[...] * BN_S) + e2[...])
    p = jax.nn.sigmoid(jnp.sum(x2 * w3t[...], axis=1, keepdims=True) + b3[...])
    out[...] = jnp.broadcast_to(p, out.shape)


def _tc_rc(ha, hb, w1, b1, g1, e1, w2, b2, g2, e2, w3t, b3):
    vspec = pl.BlockSpec(memory_space=pltpu.VMEM)
    bspec = pl.BlockSpec((512, H), lambda i: (i, 0))
    return pl.pallas_call(
        _rc_body,
        grid=(P // 512,),
        in_specs=[bspec, bspec] + [vspec] * 10,
        out_specs=bspec,
        out_shape=jax.ShapeDtypeStruct((P, H), jnp.float32),
    )(ha, hb, w1, b1, g1, e1, w2, b2, g2, e2, w3t, b3)


def kernel(x, edge_index, forward_level, backward_level, gate, forward_index,
           rc_pair_index, params):
    del x, backward_level, forward_index
    p = params
    fl = forward_level.astype(jnp.int32)
    g = gate[:, 0].astype(jnp.int32)
    src = edge_index[0].astype(jnp.int32)
    dst = edge_index[1].astype(jnp.int32)

    upd = (fl >= 1) & (g >= 1)
    gid = jnp.where(upd, (fl - 1) * 2 + (g - 1), NGROUP).astype(jnp.int32)
    ks = jnp.arange(NGROUP + 1, dtype=jnp.int32)
    onehot_n = gid[None, :] == ks[:, None]
    counts = jnp.sum(onehot_n, axis=1).astype(jnp.int32)
    occ_n = jnp.cumsum(onehot_n.astype(jnp.int32), axis=1)
    pad128 = ((counts[:NGROUP] + 127) // 128) * 128
    z1 = jnp.zeros((1,), jnp.int32)
    ns = jnp.concatenate([z1, jnp.cumsum(pad128).astype(jnp.int32)])
    rank = jnp.sum(
        jnp.where(onehot_n, ns[:, None] + occ_n - 1, 0), axis=0
    ).astype(jnp.int32)

    rc0 = rc_pair_index[0].astype(jnp.int32)
    rc1 = rc_pair_index[1].astype(jnp.int32)
    rs_e, rd_e, rc0r, rc1r = _sc_ranks(rank, src, dst, rc0, rc1)

    egid = jnp.sum((rd_e[None, :] >= ns[1:, None]).astype(jnp.int32),
                   axis=0).astype(jnp.int32)
    onehot_e = egid[None, :] == ks[:, None]
    ecounts = jnp.sum(onehot_e, axis=1).astype(jnp.int32)
    occ_e = jnp.cumsum(onehot_e.astype(jnp.int32), axis=1)
    epad = ((ecounts[:NGROUP] + CH - 1) // CH) * CH
    es = jnp.concatenate([z1, jnp.cumsum(epad).astype(jnp.int32)])
    epos = jnp.sum(
        jnp.where(onehot_e, es[:, None] + occ_e - 1, 0), axis=0
    ).astype(jnp.int32)

    jj = jnp.arange(CH, dtype=jnp.int32)[None, :]
    ec14 = ecounts[:NGROUP]
    pp = jnp.where(jj < (epad - ec14)[:, None],
                   (es[:NGROUP] + ec14)[:, None] + jj,
                   EPAD + jj).astype(jnp.int32).ravel()
    sink = (EPAD + (jnp.arange(CHB, dtype=jnp.int32) % 128)).astype(jnp.int32)
    npad_e = NGROUP * CH
    rs_ext = jnp.concatenate([rs_e, jnp.zeros((npad_e + CHB,), jnp.int32)])
    rd_ext = jnp.concatenate([rd_e, jnp.full((npad_e + CHB,), TRASH, jnp.int32)])
    pos_ext = jnp.concatenate([epos, pp, sink])

    srs, drs = _sc_slots(rs_ext, rd_ext, pos_ext)

    metas = jnp.stack([
        es[:NGROUP], epad // CH, ns[:NGROUP], pad128 // 8, pad128 // 128,
    ], axis=1).astype(jnp.int32)
    metas = jnp.pad(metas, ((0, 0), (0, 16 - metas.shape[1])))

    def row(v):
        return v.reshape(1, -1)

    aw = (p['and_w1'], row(p['and_b1']), p['and_w2'], row(p['and_b2']),
          p['and_w3'], row(p['and_b3']))
    nw = (p['not_w1'], row(p['not_b1']), p['not_w2'], row(p['not_b2']),
          p['not_w3'], row(p['not_b3']))

    hs, ma, mn = _tc_init(aw[1], aw[2], aw[3], aw[4], aw[5],
                          nw[1], nw[2], nw[3], nw[4], nw[5])

    for k in range(NGROUP):
        pre = 'and' if k % 2 == 0 else 'not'
        msg_t = ma if pre == 'and' else mn
        agg = _sc_group(msg_t, srs, drs, metas[k])
        hs, ma, mn = _tc_group(
            metas[k], agg, hs, ma, mn,
            p[pre + '_wih'], row(p[pre + '_bih']), row(p[pre + '_bhh']),
            aw, nw)

    h_out, rca, rcb = _sc_final(hs, rank, rc0r, rc1r)

    prob_full = _tc_prob(
        h_out, p['prob_w1'], row(p['prob_b1']), row(p['prob_g1']),
        row(p['prob_be1']), p['prob_w2'], row(p['prob_b2']),
        row(p['prob_g2']), row(p['prob_be2']),
        p['prob_w3'].reshape(1, -1), p['prob_b3'].reshape(1, 1))
    rc_full = _tc_rc(
        rca, rcb, p['rc_w1'], row(p['rc_b1']), row(p['rc_g1']),
        row(p['rc_be1']), p['rc_w2'], row(p['rc_b2']),
        row(p['rc_g2']), row(p['rc_be2']),
        p['rc_w3'].reshape(1, -1), p['rc_b3'].reshape(1, 1))

    return (h_out, prob_full[:, 0:1], rc_full[:, 0:1])


# speedup vs baseline: 34.8047x; 1.1841x over previous
import functools

import jax
import jax.numpy as jnp
from jax import lax
from jax.experimental import pallas as pl
from jax.experimental.pallas import tpu as pltpu
from jax.experimental.pallas import tpu_sc as plsc

N = 10000
E = 160000
P = 4096
H = 128
M = 512
NPAD = 12288
TRASH = NPAD - 1
CH = 128
EPAD = E + 14 * CH
EPADS = 163840
NGROUP = 14
NW = 32
CHA = 1280
CHB = 1024
EE = E + 14 * CH + CHB
BN_S = float(1.0 / (1.0 + 1e-5) ** 0.5)


def _iota16():
    return lax.iota(jnp.int32, 16)


def _sc_scalar(vec, j):
    return jnp.sum(jnp.where(_iota16() == j, vec, 0))


def _sc_ranks_body(rank_r, src_r, dst_r, rc0_r, rc1_r,
                   rs_o, rd_o, r0_o, r1_o,
                   rank_v, cbuf, obuf, cbuf1, obuf1, sem):
    del sem
    cid = lax.axis_index("c")
    sid = lax.axis_index("s")
    w = sid * 2 + cid
    pltpu.sync_copy(rank_r, rank_v)

    def gath(in_hbm, out_hbm, t, ib, ob, chp):
        off = pl.multiple_of(t * chp, 8)
        pltpu.sync_copy(in_hbm.at[pl.ds(off, chp)], ib)

        def it(i, carry):
            o = pl.multiple_of(i * 16, 8)
            idx = ib[pl.ds(o, 16)]
            ob[pl.ds(o, 16)] = plsc.load_gather(rank_v, [idx])
            return carry

        lax.fori_loop(0, chp // 16, it, 0)
        pltpu.sync_copy(ob, out_hbm.at[pl.ds(off, chp)])

    def body(q, carry):
        t = w + q * NW
        gath(src_r, rs_o, t, cbuf, obuf, CHA)
        gath(dst_r, rd_o, t, cbuf, obuf, CHA)
        return carry

    lax.fori_loop(0, (E // CHA + NW - 1 - w) // NW, body, 0)
    gath(rc0_r, r0_o, w, cbuf1, obuf1, 128)
    gath(rc1_r, r1_o, w, cbuf1, obuf1, 128)


_sc_ranks = functools.partial(
    pl.kernel,
    out_type=(
        jax.ShapeDtypeStruct((E,), jnp.int32),
        jax.ShapeDtypeStruct((E,), jnp.int32),
        jax.ShapeDtypeStruct((P,), jnp.int32),
        jax.ShapeDtypeStruct((P,), jnp.int32),
    ),
    mesh=plsc.VectorSubcoreMesh(core_axis_name="c", subcore_axis_name="s"),
    compiler_params=pltpu.CompilerParams(needs_layout_passes=False),
    scratch_types=[
        pltpu.VMEM((N,), jnp.int32),
        pltpu.VMEM((CHA,), jnp.int32),
        pltpu.VMEM((CHA,), jnp.int32),
        pltpu.VMEM((128,), jnp.int32),
        pltpu.VMEM((128,), jnp.int32),
        pltpu.SemaphoreType.DMA,
    ],
)(_sc_ranks_body)


def _sc_slots_body(rs_r, rd_r, pos_r, srs_o, drs_o,
                   vbuf, pbuf, zbuf, sbuf, sem):
    del sem
    cid = lax.axis_index("c")
    sid = lax.axis_index("s")
    for i in range(CHB // 16):
        zbuf[pl.ds(i * 16, 16)] = jnp.zeros((16,), jnp.int32)

    def zb(t, carry):
        off = pl.multiple_of((sid + t * 16) * CHB, 8)
        pltpu.sync_copy(zbuf, sbuf.at[pl.ds(off, CHB)])
        return carry

    lax.fori_loop(0, (EPADS // CHB + 15 - sid) // 16, zb, 0)
    plsc.subcore_barrier()

    def scat(val_r):
        def body(q, carry):
            off = pl.multiple_of((sid + q * 16) * CHB, 8)
            pltpu.sync_copy(pos_r.at[pl.ds(off, CHB)], pbuf)
            pltpu.sync_copy(val_r.at[pl.ds(off, CHB)], vbuf)
            pltpu.sync_copy(vbuf, sbuf.at[pbuf], add=True)
            return carry

        lax.fori_loop(0, (EE // CHB + 15 - sid) // 16, body, 0)

    @pl.when(cid == 0)
    def _():
        scat(rs_r)

    @pl.when(cid == 1)
    def _():
        scat(rd_r)

    plsc.subcore_barrier()

    def out(o_r):
        def body(t, carry):
            off = pl.multiple_of((sid + t * 16) * CHB, 8)
            pltpu.sync_copy(sbuf.at[pl.ds(off, CHB)], o_r.at[pl.ds(off, CHB)])
            return carry

        lax.fori_loop(0, (EPADS // CHB + 15 - sid) // 16, body, 0)

    @pl.when(cid == 0)
    def _():
        out(srs_o)

    @pl.when(cid == 1)
    def _():
        out(drs_o)


_sc_slots = functools.partial(
    pl.kernel,
    out_type=(
        jax.ShapeDtypeStruct((EPADS,), jnp.int32),
        jax.ShapeDtypeStruct((EPADS,), jnp.int32),
    ),
    mesh=plsc.VectorSubcoreMesh(core_axis_name="c", subcore_axis_name="s"),
    compiler_params=pltpu.CompilerParams(needs_layout_passes=False),
    scratch_types=[
        pltpu.VMEM((CHB,), jnp.int32),
        pltpu.VMEM((CHB,), jnp.int32),
        pltpu.VMEM((CHB,), jnp.int32),
        pltpu.VMEM_SHARED((EPADS,), jnp.int32),
        pltpu.SemaphoreType.DMA,
    ],
)(_sc_slots_body)


def _sc_group_body(msg_t, srs_r, drs_r, meta_r, agg_out,
                   meta_v, sidx, didx, rows, zbuf, aggsh, sem):
    cid = lax.axis_index("c")
    sid = lax.axis_index("s")
    w = sid * 2 + cid
    pltpu.sync_copy(meta_r, meta_v)
    mv = meta_v[...]
    es_k = _sc_scalar(mv, 0)
    nch = _sc_scalar(mv, 1)
    ns_k = _sc_scalar(mv, 2)
    n32 = _sc_scalar(mv, 3)
    for r in range(32):
        for c in range(8):
            zbuf[r, pl.ds(c * 16, 16)] = jnp.zeros((16,), jnp.float32)

    def zbody(t, carry):
        j = pl.multiple_of(ns_k + (sid + t * 16) * 32, 8)
        pltpu.sync_copy(zbuf, aggsh.at[pl.ds(j, 32)])
        return carry

    lax.fori_loop(0, (n32 + 15 - sid) // 16, zbody, 0)
    plsc.subcore_barrier()

    def ebody(t, carry):
        off = pl.multiple_of(es_k + (w + t * NW) * CH, 8)
        pltpu.sync_copy(srs_r.at[pl.ds(off, CH)], sidx)
        pltpu.sync_copy(drs_r.at[pl.ds(off, CH)], didx)
        pltpu.async_copy(msg_t.at[sidx], rows, sem).wait()
        pltpu.sync_copy(rows, aggsh.at[didx], add=True)
        return carry

    lax.fori_loop(0, (nch + NW - 1 - w) // NW, ebody, 0)
    plsc.subcore_barrier()

    def obody(t, carry):
        j = pl.multiple_of(ns_k + (sid + t * 16) * 32, 8)
        pltpu.sync_copy(aggsh.at[pl.ds(j, 32)],
                        agg_out.at[cid, pl.ds(j, 32)])
        return carry

    lax.fori_loop(0, (n32 + 15 - sid) // 16, obody, 0)


_sc_group = functools.partial(
    pl.kernel,
    out_type=jax.ShapeDtypeStruct((2, NPAD, H), jnp.float32),
    mesh=plsc.VectorSubcoreMesh(core_axis_name="c", subcore_axis_name="s"),
    compiler_params=pltpu.CompilerParams(needs_layout_passes=False),
    scratch_types=[
        pltpu.VMEM((16,), jnp.int32),
        pltpu.VMEM((CH,), jnp.int32),
        pltpu.VMEM((CH,), jnp.int32),
        pltpu.VMEM((CH, H), jnp.float32),
        pltpu.VMEM((32, H), jnp.float32),
        pltpu.VMEM_SHARED((NPAD, H), jnp.float32),
        pltpu.SemaphoreType.DMA,
    ],
)(_sc_group_body)


def _sc_final_body(hs_r, rank_r, rc0_r, rc1_r, hout_r, rca_r, rcb_r,
                   idx80, rows80, idx128, rows128, sem):
    cid = lax.axis_index("c")
    sid = lax.axis_index("s")
    w = sid * 2 + cid

    def hb(t, carry):
        j = pl.multiple_of((w + t * NW) * 80, 8)
        pltpu.sync_copy(rank_r.at[pl.ds(j, 80)], idx80)
        pltpu.async_copy(hs_r.at[idx80], rows80, sem).wait()
        pltpu.sync_copy(rows80, hout_r.at[pl.ds(j, 80)])
        return carry

    lax.fori_loop(0, (N // 80 + NW - 1 - w) // NW, hb, 0)

    wo = pl.multiple_of(w * 128, 8)
    pltpu.sync_copy(rc0_r.at[pl.ds(wo, 128)], idx128)
    pltpu.async_copy(hs_r.at[idx128], rows128, sem).wait()
    pltpu.sync_copy(rows128, rca_r.at[pl.ds(wo, 128)])
    pltpu.sync_copy(rc1_r.at[pl.ds(wo, 128)], idx128)
    pltpu.async_copy(hs_r.at[idx128], rows128, sem).wait()
    pltpu.sync_copy(rows128, rcb_r.at[pl.ds(wo, 128)])


_sc_final = functools.partial(
    pl.kernel,
    out_type=(
        jax.ShapeDtypeStruct((N, H), jnp.float32),
        jax.ShapeDtypeStruct((P, H), jnp.float32),
        jax.ShapeDtypeStruct((P, H), jnp.float32),
    ),
    mesh=plsc.VectorSubcoreMesh(core_axis_name="c", subcore_axis_name="s"),
    compiler_params=pltpu.CompilerParams(needs_layout_passes=False),
    scratch_types=[
        pltpu.VMEM((80,), jnp.int32),
        pltpu.VMEM((80, H), jnp.float32),
        pltpu.VMEM((128,), jnp.int32),
        pltpu.VMEM((128, H), jnp.float32),
        pltpu.SemaphoreType.DMA,
    ],
)(_sc_final_body)


def _crows_body(ab1, aw2, ab2, aw3, ab3, nb1, nw2, nb2, nw3, nb3, ca_o, cn_o):
    def c_row(b1, w2, b2, w3, b3):
        x1 = jnp.broadcast_to(jax.nn.relu(b1[...]), (8, M))
        x2 = jax.nn.relu(
            jnp.dot(x1, w2[...], preferred_element_type=jnp.float32) + b2[...])
        return jnp.dot(x2, w3[...], preferred_element_type=jnp.float32) + b3[...]

    ca_o[...] = c_row(ab1, aw2, ab2, aw3, ab3)
    cn_o[...] = c_row(nb1, nw2, nb2, nw3, nb3)


def _init_body(ca, cn, hs_o, ma_o, mn_o):
    hs_o[...] = jnp.zeros((1024, H), jnp.float32)
    ma_o[...] = jnp.broadcast_to(ca[0:1, :], (1024, H))
    mn_o[...] = jnp.broadcast_to(cn[0:1, :], (1024, H))


def _tc_init(ab1, aw2, ab2, aw3, ab3, nb1, nw2, nb2, nw3, nb3):
    vspec = pl.BlockSpec(memory_space=pltpu.VMEM)
    ca, cn = pl.pallas_call(
        _crows_body,
        in_specs=[vspec] * 10,
        out_specs=[vspec] * 2,
        out_shape=[jax.ShapeDtypeStruct((8, H), jnp.float32)] * 2,
    )(ab1, aw2, ab2, aw3, ab3, nb1, nw2, nb2, nw3, nb3)
    ospec = pl.BlockSpec((1024, H), lambda i: (i, 0))
    oshape = jax.ShapeDtypeStruct((NPAD, H), jnp.float32)
    return pl.pallas_call(
        _init_body,
        grid=(NPAD // 1024,),
        in_specs=[vspec] * 2,
        out_specs=[ospec] * 3,
        out_shape=[oshape] * 3,
    )(ca, cn)


def _group_body(meta, agg, hs_i, ma_i, mn_i,
                wih, bih, bhh,
                aw1, ab1, aw2, ab2, aw3, ab3,
                nw1, nb1, nw2, nb2, nw3, nb3,
                hs_o, ma_o, mn_o,
                abuf, hbuf, mabuf, mnbuf, isem, osem):
    del hs_i, ma_i, mn_i
    base = meta[2]
    ntile = meta[4]

    def mlp3(x, w1, b1, w2, b2, w3, b3):
        x = jax.nn.relu(
            jnp.dot(x, w1[...], preferred_element_type=jnp.float32) + b1[...])
        x = jax.nn.relu(
            jnp.dot(x, w2[...], preferred_element_type=jnp.float32) + b2[...])
        return jnp.dot(x, w3[...], preferred_element_type=jnp.float32) + b3[...]

    def in_copies(t, b):
        off = base + t * 128
        return (pltpu.make_async_copy(agg.at[0, pl.ds(off, 128), :],
                                      abuf.at[b, 0], isem),
                pltpu.make_async_copy(agg.at[1, pl.ds(off, 128), :],
                                      abuf.at[b, 1], isem))

    def out_copies(t, b):
        off = base + t * 128
        return (pltpu.make_async_copy(hbuf.at[b], hs_o.at[pl.ds(off, 128)], osem),
                pltpu.make_async_copy(mabuf.at[b], ma_o.at[pl.ds(off, 128)], osem),
                pltpu.make_async_copy(mnbuf.at[b], mn_o.at[pl.ds(off, 128)], osem))

    @pl.when(ntile > 0)
    def _():
        for c in in_copies(0, 0):
            c.start()

    def tile(t, carry):
        b = lax.rem(t, 2)
        for c in in_copies(t, b):
            c.wait()

        @pl.when(t + 1 < ntile)
        def _():
            for c in in_copies(t + 1, 1 - b):
                c.start()

        a = abuf[b, 0] + abuf[b, 1]
        gi = jnp.dot(a, wih[...], preferred_element_type=jnp.float32) + bih[...]
        bh = bhh[...]
        r = jax.nn.sigmoid(gi[:, :H] + bh[:, :H])
        z = jax.nn.sigmoid(gi[:, H:2 * H] + bh[:, H:2 * H])
        nn = jnp.tanh(gi[:, 2 * H:] + r * bh[:, 2 * H:])
        h = (1.0 - z) * nn

        @pl.when(t >= 2)
        def _():
            for c in out_copies(t, b):
                c.wait()

        hbuf[b] = h
        mabuf[b] = mlp3(h, aw1, ab1, aw2, ab2, aw3, ab3)
        mnbuf[b] = mlp3(h, nw1, nb1, nw2, nb2, nw3, nb3)
        for c in out_copies(t, b):
            c.start()
        return carry

    lax.fori_loop(0, ntile, tile, 0)

    @pl.when(ntile >= 2)
    def _():
        for c in out_copies(0, 0):
            c.wait()

    @pl.when(ntile >= 1)
    def _():
        for c in out_copies(0, 0):
            c.wait()


def _tc_group(meta, agg, hs, ma, mn, wih, bih, bhh, aw, nw):
    aspec = pl.BlockSpec(memory_space=pl.ANY)
    vspec = pl.BlockSpec(memory_space=pltpu.VMEM)
    sspec = pl.BlockSpec(memory_space=pltpu.SMEM)
    oshape = jax.ShapeDtypeStruct((NPAD, H), jnp.float32)
    return pl.pallas_call(
        _group_body,
        in_specs=[sspec, aspec, aspec, aspec, aspec] + [vspec] * 15,
        out_specs=[aspec] * 3,
        out_shape=[oshape] * 3,
        input_output_aliases={2: 0, 3: 1, 4: 2},
        scratch_shapes=[
            pltpu.VMEM((2, 2, 128, H), jnp.float32),
            pltpu.VMEM((2, 128, H), jnp.float32),
            pltpu.VMEM((2, 128, H), jnp.float32),
            pltpu.VMEM((2, 128, H), jnp.float32),
            pltpu.SemaphoreType.DMA,
            pltpu.SemaphoreType.DMA,
        ],
    )(meta, agg, hs, ma, mn, wih, bih, bhh, *aw, *nw)


def _prob_body(h, w1, b1, g1, e1, w2, b2, g2, e2, w3t, b3, out):
    y1 = jnp.dot(h[...], w1[...], preferred_element_type=jnp.float32) + b1[...]
    x1 = jax.nn.relu(y1 * (g1[...] * BN_S) + e1[...])
    y2 = jnp.dot(x1, w2[...], preferred_element_type=jnp.float32) + b2[...]
    x2 = jax.nn.relu(y2 * (g2[...] * BN_S) + e2[...])
    p = jnp.sum(x2 * w3t[...], axis=1, keepdims=True) + b3[...]
    out[...] = jnp.broadcast_to(p, out.shape)


def _tc_prob(h, w1, b1, g1, e1, w2, b2, g2, e2, w3t, b3):
    vspec = pl.BlockSpec(memory_space=pltpu.VMEM)
    return pl.pallas_call(
        _prob_body,
        grid=(25,),
        in_specs=[pl.BlockSpec((400, H), lambda i: (i, 0))] + [vspec] * 10,
        out_specs=pl.BlockSpec((400, H), lambda i: (i, 0)),
        out_shape=jax.ShapeDtypeStruct((N, H), jnp.float32),
    )(h, w1, b1, g1, e1, w2, b2, g2, e2, w3t, b3)


def _rc_body(ha, hb, w1, b1, g1, e1, w2, b2, g2, e2, w3t, b3, out):
    y1 = (jnp.dot(ha[...], w1[0:H], preferred_element_type=jnp.float32)
          + jnp.dot(hb[...], w1[H:2 * H], preferred_element_type=jnp.float32)
          + b1[...])
    x1 = jax.nn.relu(y1 * (g1[...] * BN_S) + e1[...])
    y2 = jnp.dot(x1, w2[...], preferred_element_type=jnp.float32) + b2[...]
    x2 = jax.nn.relu(y2 * (g2[...] * BN_S) + e2[...])
    p = jax.nn.sigmoid(jnp.sum(x2 * w3t[...], axis=1, keepdims=True) + b3[...])
    out[...] = jnp.broadcast_to(p, out.shape)


def _tc_rc(ha, hb, w1, b1, g1, e1, w2, b2, g2, e2, w3t, b3):
    vspec = pl.BlockSpec(memory_space=pltpu.VMEM)
    bspec = pl.BlockSpec((512, H), lambda i: (i, 0))
    return pl.pallas_call(
        _rc_body,
        grid=(P // 512,),
        in_specs=[bspec, bspec] + [vspec] * 10,
        out_specs=bspec,
        out_shape=jax.ShapeDtypeStruct((P, H), jnp.float32),
    )(ha, hb, w1, b1, g1, e1, w2, b2, g2, e2, w3t, b3)


def kernel(x, edge_index, forward_level, backward_level, gate, forward_index,
           rc_pair_index, params):
    del x, backward_level, forward_index
    p = params
    fl = forward_level.astype(jnp.int32)
    g = gate[:, 0].astype(jnp.int32)
    src = edge_index[0].astype(jnp.int32)
    dst = edge_index[1].astype(jnp.int32)

    upd = (fl >= 1) & (g >= 1)
    gid = jnp.where(upd, (fl - 1) * 2 + (g - 1), NGROUP).astype(jnp.int32)
    ks = jnp.arange(NGROUP + 1, dtype=jnp.int32)
    onehot_n = gid[None, :] == ks[:, None]
    counts = jnp.sum(onehot_n, axis=1).astype(jnp.int32)
    occ_n = jnp.cumsum(onehot_n.astype(jnp.int32), axis=1)
    pad128 = ((counts[:NGROUP] + 127) // 128) * 128
    z1 = jnp.zeros((1,), jnp.int32)
    ns = jnp.concatenate([z1, jnp.cumsum(pad128).astype(jnp.int32)])
    rank = jnp.sum(
        jnp.where(onehot_n, ns[:, None] + occ_n - 1, 0), axis=0
    ).astype(jnp.int32)

    rc0 = rc_pair_index[0].astype(jnp.int32)
    rc1 = rc_pair_index[1].astype(jnp.int32)
    rs_e, rd_e, rc0r, rc1r = _sc_ranks(rank, src, dst, rc0, rc1)

    egid = jnp.sum((rd_e[None, :] >= ns[1:, None]).astype(jnp.int32),
                   axis=0).astype(jnp.int32)
    onehot_e = egid[None, :] == ks[:, None]
    ecounts = jnp.sum(onehot_e, axis=1).astype(jnp.int32)
    occ_e = jnp.cumsum(onehot_e.astype(jnp.int32), axis=1)
    epad = ((ecounts[:NGROUP] + CH - 1) // CH) * CH
    es = jnp.concatenate([z1, jnp.cumsum(epad).astype(jnp.int32)])
    epos = jnp.sum(
        jnp.where(onehot_e, es[:, None] + occ_e - 1, 0), axis=0
    ).astype(jnp.int32)

    jj = jnp.arange(CH, dtype=jnp.int32)[None, :]
    ec14 = ecounts[:NGROUP]
    pp = jnp.where(jj < (epad - ec14)[:, None],
                   (es[:NGROUP] + ec14)[:, None] + jj,
                   EPAD + jj).astype(jnp.int32).ravel()
    sink = (EPAD + (jnp.arange(CHB, dtype=jnp.int32) % 128)).astype(jnp.int32)
    npad_e = NGROUP * CH
    rs_ext = jnp.concatenate([rs_e, jnp.zeros((npad_e + CHB,), jnp.int32)])
    rd_ext = jnp.concatenate([rd_e, jnp.full((npad_e + CHB,), TRASH, jnp.int32)])
    pos_ext = jnp.concatenate([epos, pp, sink])

    srs, drs = _sc_slots(rs_ext, rd_ext, pos_ext)

    metas = jnp.stack([
        es[:NGROUP], epad // CH, ns[:NGROUP], pad128 // 32, pad128 // 128,
    ], axis=1).astype(jnp.int32)
    metas = jnp.pad(metas, ((0, 0), (0, 16 - metas.shape[1])))

    def row(v):
        return v.reshape(1, -1)

    aw = (p['and_w1'], row(p['and_b1']), p['and_w2'], row(p['and_b2']),
          p['and_w3'], row(p['and_b3']))
    nw = (p['not_w1'], row(p['not_b1']), p['not_w2'], row(p['not_b2']),
          p['not_w3'], row(p['not_b3']))

    hs, ma, mn = _tc_init(aw[1], aw[2], aw[3], aw[4], aw[5],
                          nw[1], nw[2], nw[3], nw[4], nw[5])

    for k in range(NGROUP):
        pre = 'and' if k % 2 == 0 else 'not'
        msg_t = ma if pre == 'and' else mn
        agg = _sc_group(msg_t, srs, drs, metas[k])
        hs, ma, mn = _tc_group(
            metas[k], agg, hs, ma, mn,
            p[pre + '_wih'], row(p[pre + '_bih']), row(p[pre + '_bhh']),
            aw, nw)

    h_out, rca, rcb = _sc_final(hs, rank, rc0r, rc1r)

    prob_full = _tc_prob(
        h_out, p['prob_w1'], row(p['prob_b1']), row(p['prob_g1']),
        row(p['prob_be1']), p['prob_w2'], row(p['prob_b2']),
        row(p['prob_g2']), row(p['prob_be2']),
        p['prob_w3'].reshape(1, -1), p['prob_b3'].reshape(1, 1))
    rc_full = _tc_rc(
        rca, rcb, p['rc_w1'], row(p['rc_b1']), row(p['rc_g1']),
        row(p['rc_be1']), p['rc_w2'], row(p['rc_b2']),
        row(p['rc_g2']), row(p['rc_be2']),
        p['rc_w3'].reshape(1, -1), p['rc_b3'].reshape(1, 1))

    return (h_out, prob_full[:, 0:1], rc_full[:, 0:1])


# speedup vs baseline: 37.4180x; 1.0751x over previous
import functools

import jax
import jax.numpy as jnp
from jax import lax
from jax.experimental import pallas as pl
from jax.experimental.pallas import tpu as pltpu
from jax.experimental.pallas import tpu_sc as plsc

N = 10000
E = 160000
P = 4096
H = 128
M = 512
NPAD = 11904
TRASH = NPAD - 1
CH = 128
EPAD = E + 14 * CH
EPADS = 163840
NGROUP = 14
NW = 32
CHA = 1280
CHB = 1024
EE = E + 14 * CH + CHB
BN_S = float(1.0 / (1.0 + 1e-5) ** 0.5)


def _iota16():
    return lax.iota(jnp.int32, 16)


def _sc_scalar(vec, j):
    return jnp.sum(jnp.where(_iota16() == j, vec, 0))


def _sc_ranks_body(rank_r, src_r, dst_r, rc0_r, rc1_r,
                   rs_o, rd_o, r0_o, r1_o,
                   rank_v, cbuf, obuf, cbuf1, obuf1, sem):
    del sem
    cid = lax.axis_index("c")
    sid = lax.axis_index("s")
    w = sid * 2 + cid
    pltpu.sync_copy(rank_r, rank_v)

    def gath(in_hbm, out_hbm, t, ib, ob, chp):
        off = pl.multiple_of(t * chp, 8)
        pltpu.sync_copy(in_hbm.at[pl.ds(off, chp)], ib)

        def it(i, carry):
            o = pl.multiple_of(i * 16, 8)
            idx = ib[pl.ds(o, 16)]
            ob[pl.ds(o, 16)] = plsc.load_gather(rank_v, [idx])
            return carry

        lax.fori_loop(0, chp // 16, it, 0)
        pltpu.sync_copy(ob, out_hbm.at[pl.ds(off, chp)])

    def body(q, carry):
        t = w + q * NW
        gath(src_r, rs_o, t, cbuf, obuf, CHA)
        gath(dst_r, rd_o, t, cbuf, obuf, CHA)
        return carry

    lax.fori_loop(0, (E // CHA + NW - 1 - w) // NW, body, 0)
    gath(rc0_r, r0_o, w, cbuf1, obuf1, 128)
    gath(rc1_r, r1_o, w, cbuf1, obuf1, 128)


_sc_ranks = functools.partial(
    pl.kernel,
    out_type=(
        jax.ShapeDtypeStruct((E,), jnp.int32),
        jax.ShapeDtypeStruct((E,), jnp.int32),
        jax.ShapeDtypeStruct((P,), jnp.int32),
        jax.ShapeDtypeStruct((P,), jnp.int32),
    ),
    mesh=plsc.VectorSubcoreMesh(core_axis_name="c", subcore_axis_name="s"),
    compiler_params=pltpu.CompilerParams(needs_layout_passes=False),
    scratch_types=[
        pltpu.VMEM((N,), jnp.int32),
        pltpu.VMEM((CHA,), jnp.int32),
        pltpu.VMEM((CHA,), jnp.int32),
        pltpu.VMEM((128,), jnp.int32),
        pltpu.VMEM((128,), jnp.int32),
        pltpu.SemaphoreType.DMA,
    ],
)(_sc_ranks_body)


def _sc_slots_body(rs_r, rd_r, pos_r, srs_o, drs_o,
                   vbuf, pbuf, zbuf, sbuf, sem):
    del sem
    cid = lax.axis_index("c")
    sid = lax.axis_index("s")
    for i in range(CHB // 16):
        zbuf[pl.ds(i * 16, 16)] = jnp.zeros((16,), jnp.int32)

    def zb(t, carry):
        off = pl.multiple_of((sid + t * 16) * CHB, 8)
        pltpu.sync_copy(zbuf, sbuf.at[pl.ds(off, CHB)])
        return carry

    lax.fori_loop(0, (EPADS // CHB + 15 - sid) // 16, zb, 0)
    plsc.subcore_barrier()

    def scat(val_r):
        def body(q, carry):
            off = pl.multiple_of((sid + q * 16) * CHB, 8)
            pltpu.sync_copy(pos_r.at[pl.ds(off, CHB)], pbuf)
            pltpu.sync_copy(val_r.at[pl.ds(off, CHB)], vbuf)
            pltpu.sync_copy(vbuf, sbuf.at[pbuf], add=True)
            return carry

        lax.fori_loop(0, (EE // CHB + 15 - sid) // 16, body, 0)

    @pl.when(cid == 0)
    def _():
        scat(rs_r)

    @pl.when(cid == 1)
    def _():
        scat(rd_r)

    plsc.subcore_barrier()

    def out(o_r):
        def body(t, carry):
            off = pl.multiple_of((sid + t * 16) * CHB, 8)
            pltpu.sync_copy(sbuf.at[pl.ds(off, CHB)], o_r.at[pl.ds(off, CHB)])
            return carry

        lax.fori_loop(0, (EPADS // CHB + 15 - sid) // 16, body, 0)

    @pl.when(cid == 0)
    def _():
        out(srs_o)

    @pl.when(cid == 1)
    def _():
        out(drs_o)


_sc_slots = functools.partial(
    pl.kernel,
    out_type=(
        jax.ShapeDtypeStruct((EPADS,), jnp.int32),
        jax.ShapeDtypeStruct((EPADS,), jnp.int32),
    ),
    mesh=plsc.VectorSubcoreMesh(core_axis_name="c", subcore_axis_name="s"),
    compiler_params=pltpu.CompilerParams(needs_layout_passes=False),
    scratch_types=[
        pltpu.VMEM((CHB,), jnp.int32),
        pltpu.VMEM((CHB,), jnp.int32),
        pltpu.VMEM((CHB,), jnp.int32),
        pltpu.VMEM_SHARED((EPADS,), jnp.int32),
        pltpu.SemaphoreType.DMA,
    ],
)(_sc_slots_body)


def _sc_group_body(msg_t, srs_r, drs_r, meta_r, agg_out,
                   meta_v, sidx, didx, rows, zbuf, aggsh, sem):
    cid = lax.axis_index("c")
    sid = lax.axis_index("s")
    w = sid * 2 + cid
    pltpu.sync_copy(meta_r, meta_v)
    mv = meta_v[...]
    es_k = _sc_scalar(mv, 0)
    nch = _sc_scalar(mv, 1)
    ns_k = _sc_scalar(mv, 2)
    n32 = _sc_scalar(mv, 3)
    for r in range(8):
        for c in range(8):
            zbuf[r, pl.ds(c * 16, 16)] = jnp.zeros((16,), jnp.float32)

    n8 = n32 * 4

    def zbody(t, carry):
        j = pl.multiple_of(ns_k + (sid + t * 16) * 8, 8)
        pltpu.sync_copy(zbuf, aggsh.at[pl.ds(j, 8)])
        return carry

    lax.fori_loop(0, (n8 + 15 - sid) // 16, zbody, 0)
    plsc.subcore_barrier()

    nchw = (nch + NW - 1 - w) // NW
    isem, gsem, ssem = sem

    def start_idx(t, b):
        off = pl.multiple_of(es_k + (w + t * NW) * CH, 8)
        pltpu.async_copy(srs_r.at[pl.ds(off, CH)], sidx.at[b], isem)
        pltpu.async_copy(drs_r.at[pl.ds(off, CH)], didx.at[b], isem)

    def drain_idx(b):
        pltpu.make_async_copy(srs_r.at[pl.ds(0, CH)], sidx.at[b], isem).wait()
        pltpu.make_async_copy(drs_r.at[pl.ds(0, CH)], didx.at[b], isem).wait()

    def drain_scat(b):
        pltpu.make_async_copy(rows.at[b], aggsh.at[pl.ds(0, CH)], ssem).wait()


    @pl.when(nchw > 0)
    def _():
        start_idx(0, 0)

    def ebody(t, carry):
        b = lax.rem(t, 3)
        b2 = lax.rem(t, 2)
        drain_idx(b)

        @pl.when(t >= 2)
        def _():
            drain_scat(b2)

        @pl.when(t + 1 < nchw)
        def _():
            start_idx(t + 1, lax.rem(t + 1, 3))

        pltpu.async_copy(msg_t.at[sidx.at[b]], rows.at[b2], gsem).wait()
        pltpu.async_copy(rows.at[b2], aggsh.at[didx.at[b]], ssem, add=True)
        return carry

    lax.fori_loop(0, nchw, ebody, 0)

    @pl.when(nchw >= 2)
    def _():
        drain_scat(0)

    @pl.when(nchw >= 1)
    def _():
        drain_scat(0)

    plsc.subcore_barrier()

    def obody(t, carry):
        j = pl.multiple_of(ns_k + (sid + t * 16) * 32, 8)
        pltpu.sync_copy(aggsh.at[pl.ds(j, 32)],
                        agg_out.at[cid, pl.ds(j, 32)])
        return carry

    lax.fori_loop(0, (n32 + 15 - sid) // 16, obody, 0)


_sc_group = functools.partial(
    pl.kernel,
    out_type=jax.ShapeDtypeStruct((2, NPAD, H), jnp.float32),
    mesh=plsc.VectorSubcoreMesh(core_axis_name="c", subcore_axis_name="s"),
    compiler_params=pltpu.CompilerParams(needs_layout_passes=False),
    scratch_types=[
        pltpu.VMEM((16,), jnp.int32),
        pltpu.VMEM((3, CH), jnp.int32),
        pltpu.VMEM((3, CH), jnp.int32),
        pltpu.VMEM((2, CH, H), jnp.float32),
        pltpu.VMEM((8, H), jnp.float32),
        pltpu.VMEM_SHARED((NPAD, H), jnp.float32),
        (pltpu.SemaphoreType.DMA, pltpu.SemaphoreType.DMA,
         pltpu.SemaphoreType.DMA),
    ],
)(_sc_group_body)


def _sc_final_body(hs_r, rank_r, rc0_r, rc1_r, hout_r, rca_r, rcb_r,
                   idx80, rows80, idx128, rows128, sem):
    cid = lax.axis_index("c")
    sid = lax.axis_index("s")
    w = sid * 2 + cid

    def hb(t, carry):
        j = pl.multiple_of((w + t * NW) * 80, 8)
        pltpu.sync_copy(rank_r.at[pl.ds(j, 80)], idx80)
        pltpu.async_copy(hs_r.at[idx80], rows80, sem).wait()
        pltpu.sync_copy(rows80, hout_r.at[pl.ds(j, 80)])
        return carry

    lax.fori_loop(0, (N // 80 + NW - 1 - w) // NW, hb, 0)

    wo = pl.multiple_of(w * 128, 8)
    pltpu.sync_copy(rc0_r.at[pl.ds(wo, 128)], idx128)
    pltpu.async_copy(hs_r.at[idx128], rows128, sem).wait()
    pltpu.sync_copy(rows128, rca_r.at[pl.ds(wo, 128)])
    pltpu.sync_copy(rc1_r.at[pl.ds(wo, 128)], idx128)
    pltpu.async_copy(hs_r.at[idx128], rows128, sem).wait()
    pltpu.sync_copy(rows128, rcb_r.at[pl.ds(wo, 128)])


_sc_final = functools.partial(
    pl.kernel,
    out_type=(
        jax.ShapeDtypeStruct((N, H), jnp.float32),
        jax.ShapeDtypeStruct((P, H), jnp.float32),
        jax.ShapeDtypeStruct((P, H), jnp.float32),
    ),
    mesh=plsc.VectorSubcoreMesh(core_axis_name="c", subcore_axis_name="s"),
    compiler_params=pltpu.CompilerParams(needs_layout_passes=False),
    scratch_types=[
        pltpu.VMEM((80,), jnp.int32),
        pltpu.VMEM((80, H), jnp.float32),
        pltpu.VMEM((128,), jnp.int32),
        pltpu.VMEM((128, H), jnp.float32),
        pltpu.SemaphoreType.DMA,
    ],
)(_sc_final_body)


def _crows_body(ab1, aw2, ab2, aw3, ab3, nb1, nw2, nb2, nw3, nb3, ca_o, cn_o):
    def c_row(b1, w2, b2, w3, b3):
        x1 = jnp.broadcast_to(jax.nn.relu(b1[...]), (8, M))
        x2 = jax.nn.relu(
            jnp.dot(x1, w2[...], preferred_element_type=jnp.float32) + b2[...])
        return jnp.dot(x2, w3[...], preferred_element_type=jnp.float32) + b3[...]

    ca_o[...] = c_row(ab1, aw2, ab2, aw3, ab3)
    cn_o[...] = c_row(nb1, nw2, nb2, nw3, nb3)


def _init_body(ca, cn, hs_o, ma_o, mn_o):
    hs_o[...] = jnp.zeros((384, H), jnp.float32)
    ma_o[...] = jnp.broadcast_to(ca[0:1, :], (384, H))
    mn_o[...] = jnp.broadcast_to(cn[0:1, :], (384, H))


def _tc_init(ab1, aw2, ab2, aw3, ab3, nb1, nw2, nb2, nw3, nb3):
    vspec = pl.BlockSpec(memory_space=pltpu.VMEM)
    ca, cn = pl.pallas_call(
        _crows_body,
        in_specs=[vspec] * 10,
        out_specs=[vspec] * 2,
        out_shape=[jax.ShapeDtypeStruct((8, H), jnp.float32)] * 2,
    )(ab1, aw2, ab2, aw3, ab3, nb1, nw2, nb2, nw3, nb3)
    ospec = pl.BlockSpec((384, H), lambda i: (i, 0))
    oshape = jax.ShapeDtypeStruct((NPAD, H), jnp.float32)
    return pl.pallas_call(
        _init_body,
        grid=(NPAD // 384,),
        in_specs=[vspec] * 2,
        out_specs=[ospec] * 3,
        out_shape=[oshape] * 3,
    )(ca, cn)


def _group_body(meta, agg, hs_i, ma_i, mn_i,
                wih, bih, bhh,
                aw1, ab1, aw2, ab2, aw3, ab3,
                nw1, nb1, nw2, nb2, nw3, nb3,
                hs_o, ma_o, mn_o,
                abuf, hbuf, mabuf, mnbuf, isem, osem):
    del hs_i, ma_i, mn_i
    base = meta[2]
    ntile = meta[4]

    def mlp3(x, w1, b1, w2, b2, w3, b3):
        x = jax.nn.relu(
            jnp.dot(x, w1[...], preferred_element_type=jnp.float32) + b1[...])
        x = jax.nn.relu(
            jnp.dot(x, w2[...], preferred_element_type=jnp.float32) + b2[...])
        return jnp.dot(x, w3[...], preferred_element_type=jnp.float32) + b3[...]

    def in_copies(t, b):
        off = base + t * 128
        return (pltpu.make_async_copy(agg.at[0, pl.ds(off, 128), :],
                                      abuf.at[b, 0], isem),
                pltpu.make_async_copy(agg.at[1, pl.ds(off, 128), :],
                                      abuf.at[b, 1], isem))

    def out_copies(t, b):
        off = base + t * 128
        return (pltpu.make_async_copy(hbuf.at[b], hs_o.at[pl.ds(off, 128)], osem),
                pltpu.make_async_copy(mabuf.at[b], ma_o.at[pl.ds(off, 128)], osem),
                pltpu.make_async_copy(mnbuf.at[b], mn_o.at[pl.ds(off, 128)], osem))

    @pl.when(ntile > 0)
    def _():
        for c in in_copies(0, 0):
            c.start()

    def tile(t, carry):
        b = lax.rem(t, 2)
        for c in in_copies(t, b):
            c.wait()

        @pl.when(t + 1 < ntile)
        def _():
            for c in in_copies(t + 1, 1 - b):
                c.start()

        a = abuf[b, 0] + abuf[b, 1]
        gi = jnp.dot(a, wih[...], preferred_element_type=jnp.float32) + bih[...]
        bh = bhh[...]
        r = jax.nn.sigmoid(gi[:, :H] + bh[:, :H])
        z = jax.nn.sigmoid(gi[:, H:2 * H] + bh[:, H:2 * H])
        nn = jnp.tanh(gi[:, 2 * H:] + r * bh[:, 2 * H:])
        h = (1.0 - z) * nn

        @pl.when(t >= 2)
        def _():
            for c in out_copies(t, b):
                c.wait()

        hbuf[b] = h
        mabuf[b] = mlp3(h, aw1, ab1, aw2, ab2, aw3, ab3)
        mnbuf[b] = mlp3(h, nw1, nb1, nw2, nb2, nw3, nb3)
        for c in out_copies(t, b):
            c.start()
        return carry

    lax.fori_loop(0, ntile, tile, 0)

    @pl.when(ntile >= 2)
    def _():
        for c in out_copies(0, 0):
            c.wait()

    @pl.when(ntile >= 1)
    def _():
        for c in out_copies(0, 0):
            c.wait()


def _tc_group(meta, agg, hs, ma, mn, wih, bih, bhh, aw, nw):
    aspec = pl.BlockSpec(memory_space=pl.ANY)
    vspec = pl.BlockSpec(memory_space=pltpu.VMEM)
    sspec = pl.BlockSpec(memory_space=pltpu.SMEM)
    oshape = jax.ShapeDtypeStruct((NPAD, H), jnp.float32)
    return pl.pallas_call(
        _group_body,
        in_specs=[sspec, aspec, aspec, aspec, aspec] + [vspec] * 15,
        out_specs=[aspec] * 3,
        out_shape=[oshape] * 3,
        input_output_aliases={2: 0, 3: 1, 4: 2},
        scratch_shapes=[
            pltpu.VMEM((2, 2, 128, H), jnp.float32),
            pltpu.VMEM((2, 128, H), jnp.float32),
            pltpu.VMEM((2, 128, H), jnp.float32),
            pltpu.VMEM((2, 128, H), jnp.float32),
            pltpu.SemaphoreType.DMA,
            pltpu.SemaphoreType.DMA,
        ],
    )(meta, agg, hs, ma, mn, wih, bih, bhh, *aw, *nw)


def _prob_body(h, w1, b1, g1, e1, w2, b2, g2, e2, w3t, b3, out):
    y1 = jnp.dot(h[...], w1[...], preferred_element_type=jnp.float32) + b1[...]
    x1 = jax.nn.relu(y1 * (g1[...] * BN_S) + e1[...])
    y2 = jnp.dot(x1, w2[...], preferred_element_type=jnp.float32) + b2[...]
    x2 = jax.nn.relu(y2 * (g2[...] * BN_S) + e2[...])
    p = jnp.sum(x2 * w3t[...], axis=1, keepdims=True) + b3[...]
    out[...] = p


def _tc_prob(h, w1, b1, g1, e1, w2, b2, g2, e2, w3t, b3):
    vspec = pl.BlockSpec(memory_space=pltpu.VMEM)
    return pl.pallas_call(
        _prob_body,
        grid=(25,),
        in_specs=[pl.BlockSpec((400, H), lambda i: (i, 0))] + [vspec] * 10,
        out_specs=pl.BlockSpec((400, 1), lambda i: (i, 0)),
        out_shape=jax.ShapeDtypeStruct((N, 1), jnp.float32),
    )(h, w1, b1, g1, e1, w2, b2, g2, e2, w3t, b3)


def _rc_body(ha, hb, w1, b1, g1, e1, w2, b2, g2, e2, w3t, b3, out):
    y1 = (jnp.dot(ha[...], w1[0:H], preferred_element_type=jnp.float32)
          + jnp.dot(hb[...], w1[H:2 * H], preferred_element_type=jnp.float32)
          + b1[...])
    x1 = jax.nn.relu(y1 * (g1[...] * BN_S) + e1[...])
    y2 = jnp.dot(x1, w2[...], preferred_element_type=jnp.float32) + b2[...]
    x2 = jax.nn.relu(y2 * (g2[...] * BN_S) + e2[...])
    p = jax.nn.sigmoid(jnp.sum(x2 * w3t[...], axis=1, keepdims=True) + b3[...])
    out[...] = p


def _tc_rc(ha, hb, w1, b1, g1, e1, w2, b2, g2, e2, w3t, b3):
    vspec = pl.BlockSpec(memory_space=pltpu.VMEM)
    bspec = pl.BlockSpec((512, H), lambda i: (i, 0))
    return pl.pallas_call(
        _rc_body,
        grid=(P // 512,),
        in_specs=[bspec, bspec] + [vspec] * 10,
        out_specs=pl.BlockSpec((512, 1), lambda i: (i, 0)),
        out_shape=jax.ShapeDtypeStruct((P, 1), jnp.float32),
    )(ha, hb, w1, b1, g1, e1, w2, b2, g2, e2, w3t, b3)


def kernel(x, edge_index, forward_level, backward_level, gate, forward_index,
           rc_pair_index, params):
    del x, backward_level, forward_index
    p = params
    fl = forward_level.astype(jnp.int32)
    g = gate[:, 0].astype(jnp.int32)
    src = edge_index[0].astype(jnp.int32)
    dst = edge_index[1].astype(jnp.int32)

    upd = (fl >= 1) & (g >= 1)
    gid = jnp.where(upd, (fl - 1) * 2 + (g - 1), NGROUP).astype(jnp.int32)
    ks = jnp.arange(NGROUP + 1, dtype=jnp.int32)
    onehot_n = gid[None, :] == ks[:, None]
    counts = jnp.sum(onehot_n, axis=1).astype(jnp.int32)
    occ_n = jnp.cumsum(onehot_n.astype(jnp.int32), axis=1)
    pad128 = ((counts[:NGROUP] + 127) // 128) * 128
    z1 = jnp.zeros((1,), jnp.int32)
    ns = jnp.concatenate([z1, jnp.cumsum(pad128).astype(jnp.int32)])
    rank = jnp.sum(
        jnp.where(onehot_n, ns[:, None] + occ_n - 1, 0), axis=0
    ).astype(jnp.int32)

    rc0 = rc_pair_index[0].astype(jnp.int32)
    rc1 = rc_pair_index[1].astype(jnp.int32)
    rs_e, rd_e, rc0r, rc1r = _sc_ranks(rank, src, dst, rc0, rc1)

    egid = jnp.sum((rd_e[None, :] >= ns[1:, None]).astype(jnp.int32),
                   axis=0).astype(jnp.int32)
    onehot_e = egid[None, :] == ks[:, None]
    ecounts = jnp.sum(onehot_e, axis=1).astype(jnp.int32)
    occ_e = jnp.cumsum(onehot_e.astype(jnp.int32), axis=1)
    epad = ((ecounts[:NGROUP] + CH - 1) // CH) * CH
    es = jnp.concatenate([z1, jnp.cumsum(epad).astype(jnp.int32)])
    epos = jnp.sum(
        jnp.where(onehot_e, es[:, None] + occ_e - 1, 0), axis=0
    ).astype(jnp.int32)

    jj = jnp.arange(CH, dtype=jnp.int32)[None, :]
    ec14 = ecounts[:NGROUP]
    pp = jnp.where(jj < (epad - ec14)[:, None],
                   (es[:NGROUP] + ec14)[:, None] + jj,
                   EPAD + jj).astype(jnp.int32).ravel()
    sink = (EPAD + (jnp.arange(CHB, dtype=jnp.int32) % 128)).astype(jnp.int32)
    npad_e = NGROUP * CH
    rs_ext = jnp.concatenate([rs_e, jnp.zeros((npad_e + CHB,), jnp.int32)])
    rd_ext = jnp.concatenate([rd_e, jnp.full((npad_e + CHB,), TRASH, jnp.int32)])
    pos_ext = jnp.concatenate([epos, pp, sink])

    srs, drs = _sc_slots(rs_ext, rd_ext, pos_ext)

    metas = jnp.stack([
        es[:NGROUP], epad // CH, ns[:NGROUP], pad128 // 32, pad128 // 128,
    ], axis=1).astype(jnp.int32)
    metas = jnp.pad(metas, ((0, 0), (0, 16 - metas.shape[1])))

    def row(v):
        return v.reshape(1, -1)

    aw = (p['and_w1'], row(p['and_b1']), p['and_w2'], row(p['and_b2']),
          p['and_w3'], row(p['and_b3']))
    nw = (p['not_w1'], row(p['not_b1']), p['not_w2'], row(p['not_b2']),
          p['not_w3'], row(p['not_b3']))

    hs, ma, mn = _tc_init(aw[1], aw[2], aw[3], aw[4], aw[5],
                          nw[1], nw[2], nw[3], nw[4], nw[5])

    for k in range(NGROUP):
        pre = 'and' if k % 2 == 0 else 'not'
        msg_t = ma if pre == 'and' else mn
        agg = _sc_group(msg_t, srs, drs, metas[k])
        hs, ma, mn = _tc_group(
            metas[k], agg, hs, ma, mn,
            p[pre + '_wih'], row(p[pre + '_bih']), row(p[pre + '_bhh']),
            aw, nw)

    h_out, rca, rcb = _sc_final(hs, rank, rc0r, rc1r)

    prob = _tc_prob(
        h_out, p['prob_w1'], row(p['prob_b1']), row(p['prob_g1']),
        row(p['prob_be1']), p['prob_w2'], row(p['prob_b2']),
        row(p['prob_g2']), row(p['prob_be2']),
        p['prob_w3'].reshape(1, -1), p['prob_b3'].reshape(1, 1))
    is_rc = _tc_rc(
        rca, rcb, p['rc_w1'], row(p['rc_b1']), row(p['rc_g1']),
        row(p['rc_be1']), p['rc_w2'], row(p['rc_b2']),
        row(p['rc_g2']), row(p['rc_be2']),
        p['rc_w3'].reshape(1, -1), p['rc_b3'].reshape(1, 1))

    return (h_out, prob, is_rc)
